# baseline (speedup 1.0000x reference)
import functools

import jax
import jax.numpy as jnp
from jax import lax
from jax.experimental import pallas as pl
from jax.experimental.pallas import tpu as pltpu
import jax.experimental.pallas.tpu_sc as plsc

N_NODES = 262144
N_EDGES = 524288
NUM_IN = 128
NUM_HIDDEN = 256
NUM_CLASS = 10
N_GRAPHS = 8192
E_PER_GRAPH = 64

NC = 2
NS = 16
L = 16

_MESH = dict(core_axis_name="c", subcore_axis_name="s", num_cores=NC,
             num_subcores=NS)

_DEG_W = 4096
_E_PER_TILE_DEG = N_EDGES // (NC * NS)
_N_PER_TILE = N_NODES // NS


def _deg_body(index_hbm, ew_win_hbm, degp_hbm, dacc, zbuf, colbuf, ewb):
  c = lax.axis_index("c")
  s = lax.axis_index("s")

  def zero_vec(i, _):
    zbuf[pl.ds(i * L, L)] = jnp.zeros((L,), jnp.float32)
    return 0
  lax.fori_loop(0, _DEG_W // L, zero_vec, 0)

  def zero_chunk(i, _):
    pltpu.sync_copy(zbuf, dacc.at[pl.ds(s * _N_PER_TILE + i * _DEG_W, _DEG_W)])
    return 0
  lax.fori_loop(0, _N_PER_TILE // _DEG_W, zero_chunk, 0)
  plsc.subcore_barrier()

  pltpu.sync_copy(ew_win_hbm, ewb)
  tile_base = (c * NS + s) * _E_PER_TILE_DEG

  def win(w, _):
    ebase = tile_base + w * _DEG_W
    pltpu.sync_copy(index_hbm.at[1, pl.ds(ebase, _DEG_W)], colbuf)
    pltpu.sync_copy(ewb, dacc.at[colbuf], add=True)
    return 0
  lax.fori_loop(0, _E_PER_TILE_DEG // _DEG_W, win, 0)
  plsc.subcore_barrier()

  pltpu.sync_copy(dacc.at[pl.ds(s * _N_PER_TILE, _N_PER_TILE)],
                  degp_hbm.at[c, pl.ds(s * _N_PER_TILE, _N_PER_TILE)])


def _k_deg(index, ew_win):
  f = pl.kernel(
      _deg_body,
      out_type=jax.ShapeDtypeStruct((NC, N_NODES), jnp.float32),
      mesh=plsc.VectorSubcoreMesh(**_MESH),
      compiler_params=pltpu.CompilerParams(needs_layout_passes=False),
      scratch_types=[
          pltpu.VMEM_SHARED((N_NODES,), jnp.float32),
          pltpu.VMEM((_DEG_W,), jnp.float32),
          pltpu.VMEM((_DEG_W,), jnp.int32),
          pltpu.VMEM((_DEG_W,), jnp.float32),
      ],
  )
  return f(index, ew_win)


_R = 10240
_CHUNKS = 26
_PASSES = _CHUNKS // NC
_HOP_W = 2048
_UNROLL = 4
_E_PER_TILE = N_EDGES // NS
_GK = 128
_LCAP = 2208
_RPT = _R // NS
_TAIL_BASE = (_CHUNKS - 1) * _R
_TAIL_ROWS = N_NODES - _TAIL_BASE
_TAIL_RPT = _TAIL_ROWS // NS
_WPAD = 64
_LCBITS = 14
_LCMASK = (1 << _LCBITS) - 1


def _hop_finish(src_hbm, acc, st_r2, st_c2, st_w2, gbuf2, gsem, b):
  pltpu.make_async_copy(src_hbm.at[st_r2.at[b]], gbuf2.at[b], gsem).wait()

  def scale(g, _):
    wv = st_w2[b, pl.ds(g * L, L)]
    for k in range(L):
      wsp = jnp.full((L,), wv[k], jnp.float32)
      r = g * L + k
      for q in range(NUM_IN // L):
        gbuf2[b, r, pl.ds(q * L, L)] = gbuf2[b, r, pl.ds(q * L, L)] * wsp
    return 0
  lax.fori_loop(0, _GK // L, scale, 0)

  pltpu.sync_copy(gbuf2.at[b], acc.at[st_c2.at[b]], add=True)


def _hop_fire(src_hbm, acc, flr, flc, st_r2, st_c2, st_w2, gbuf2, ewb,
              gsem, start, fctr, mall):
  par = lax.rem(fctr, 2)

  def stage(i, _):
    st_r2[par, pl.ds(i * L, L)] = flr[pl.ds(start + i * L, L)]
    pk = flc[pl.ds(start + i * L, L)]
    st_c2[par, pl.ds(i * L, L)] = pk & _LCMASK
    st_w2[par, pl.ds(i * L, L)] = plsc.load_gather(
        ewb, [lax.shift_right_logical(pk, _LCBITS)], mask=mall)
    return 0
  lax.fori_loop(0, _GK // L, stage, 0)

  pltpu.async_copy(src_hbm.at[st_r2.at[par]], gbuf2.at[par], gsem)

  @pl.when(fctr >= 1)
  def _():
    _hop_finish(src_hbm, acc, st_r2, st_c2, st_w2, gbuf2, gsem, 1 - par)


def _hop_body(src_hbm, index_hbm, w64_hbm, dst_hbm,
              acc, flr, flc, st_r2, st_c2, st_w2, gbuf2, colw, roww, ewb,
              gsem, wsem):
  c = lax.axis_index("c")
  s = lax.axis_index("s")
  pltpu.sync_copy(w64_hbm, ewb)
  lane = lax.iota(jnp.int32, L)
  mall = lane >= 0
  tile_e = s * _E_PER_TILE
  n_win = _E_PER_TILE // _HOP_W

  def do_pass(p, _):
    k = NC * p + c
    is_tail = k == (_CHUNKS - 1)
    base = jnp.where(is_tail, _TAIL_BASE, k * _R)
    init_base = jnp.where(is_tail, N_NODES - _R, k * _R)
    hi = jnp.where(is_tail, N_NODES, k * _R + _R)

    pltpu.sync_copy(src_hbm.at[pl.ds(init_base + s * _RPT, _RPT)],
                    acc.at[pl.ds(s * _RPT, _RPT)])
    plsc.subcore_barrier()

    def drain(cnt, fired, fctr):
      def one(d, carry):
        fired, fctr = carry
        go = fired + _GK <= cnt
        @pl.when(go)
        def _():
          _hop_fire(src_hbm, acc, flr, flc, st_r2, st_c2, st_w2, gbuf2,
                    ewb, gsem, fired, fctr, mall)
        adv = jnp.where(go, 1, 0)
        return (fired + adv * _GK, fctr + adv)
      fired, fctr = lax.fori_loop(0, (_LCAP + _GK - 1) // _GK, one,
                                  (fired, fctr))
      rem = cnt - fired
      def mv(i, _):
        @pl.when(i * L < rem)
        def _():
          flr[pl.ds(i * L, L)] = flr[pl.ds(fired + i * L, L)]
          flc[pl.ds(i * L, L)] = flc[pl.ds(fired + i * L, L)]
        return 0
      lax.fori_loop(0, _GK // L, mv, 0)
      return rem, fctr

    def win(w, carry):
      cnt, fctr = carry
      ebase = tile_e + w * _HOP_W
      pltpu.async_copy(index_hbm.at[1, pl.ds(ebase, _HOP_W)], colw, wsem)
      pltpu.async_copy(index_hbm.at[0, pl.ds(ebase, _HOP_W)], roww, wsem)
      pltpu.make_async_copy(index_hbm.at[1, pl.ds(ebase, _HOP_W)], colw,
                            wsem).wait()
      pltpu.make_async_copy(index_hbm.at[0, pl.ds(ebase, _HOP_W)], roww,
                            wsem).wait()

      def scan(jj, cnt):
        for u in range(_UNROLL):
          j = jj * _UNROLL + u
          c16 = colw[pl.ds(j * L, L)]
          m = (c16 >= base) & (c16 < hi)
          r16 = roww[pl.ds(j * L, L)]
          widx16 = lax.rem(j, 4) * L + lane
          pk16 = (c16 - init_base) | lax.shift_left(widx16, _LCBITS)
          cs = plsc.cumsum(m.astype(jnp.int32))
          pos = cs + (cnt - 1)
          plsc.store_scatter(flr, [pos], r16, mask=m)
          plsc.store_scatter(flc, [pos], pk16, mask=m)
          cnt = cnt + cs[L - 1]
        return cnt
      cnt = lax.fori_loop(0, _HOP_W // L // _UNROLL, scan, cnt)
      return drain(cnt, 0, fctr)

    cnt, fctr = lax.fori_loop(0, n_win, win, (0, 0))

    cnt_pad = jnp.where(cnt > 0, ((cnt + _GK - 1) // _GK) * _GK, 0)
    pad_pk = lane | (_WPAD << _LCBITS)
    def pad(i, _):
      @pl.when(cnt + i * L < cnt_pad)
      def _():
        off = cnt + i * L
        flr[pl.ds(off, L)] = lane
        flc[pl.ds(off, L)] = pad_pk
      return 0
    lax.fori_loop(0, _GK // L, pad, 0)
    _, fctr = drain(cnt_pad, 0, fctr)

    @pl.when(fctr >= 1)
    def _():
      _hop_finish(src_hbm, acc, st_r2, st_c2, st_w2, gbuf2, gsem,
                  lax.rem(fctr - 1, 2))

    plsc.subcore_barrier()
    @pl.when(jnp.logical_not(is_tail))
    def _():
      pltpu.sync_copy(acc.at[pl.ds(s * _RPT, _RPT)],
                      dst_hbm.at[pl.ds(init_base + s * _RPT, _RPT)])
    @pl.when(is_tail)
    def _():
      pltpu.sync_copy(
          acc.at[pl.ds(_R - _TAIL_ROWS + s * _TAIL_RPT, _TAIL_RPT)],
          dst_hbm.at[pl.ds(_TAIL_BASE + s * _TAIL_RPT, _TAIL_RPT)])
    plsc.subcore_barrier()
    return 0

  lax.fori_loop(0, _PASSES, do_pass, 0)


def _k_hop(src, index, w64pad):
  f = pl.kernel(
      _hop_body,
      out_type=jax.ShapeDtypeStruct((N_NODES, NUM_IN), jnp.float32),
      mesh=plsc.VectorSubcoreMesh(**_MESH),
      compiler_params=pltpu.CompilerParams(needs_layout_passes=False),
      scratch_types=[
          pltpu.VMEM_SHARED((_R, NUM_IN), jnp.float32),
          pltpu.VMEM((_LCAP,), jnp.int32),
          pltpu.VMEM((_LCAP,), jnp.int32),
          pltpu.VMEM((2, _GK), jnp.int32),
          pltpu.VMEM((2, _GK), jnp.int32),
          pltpu.VMEM((2, _GK), jnp.float32),
          pltpu.VMEM((2, _GK, NUM_IN), jnp.float32),
          pltpu.VMEM((_HOP_W,), jnp.int32),
          pltpu.VMEM((_HOP_W,), jnp.int32),
          pltpu.VMEM((_WPAD + 16,), jnp.float32),
          pltpu.SemaphoreType.DMA,
          pltpu.SemaphoreType.DMA,
      ],
  )
  return f(src, index, w64pad)


_POOL_W = 2048
_ROWS_PER_TILE = N_NODES // (NC * NS)
_G_PER_TILE = N_GRAPHS // NS
LG = 16


_PACC = N_GRAPHS * LG
_ZP = _PACC // NS


def _pool_body(lg_hbm, batch_hbm, out_hbm, pacc, zbuf, rbuf, bbuf, ibuf):
  c = lax.axis_index("c")
  s = lax.axis_index("s")
  lane = lax.iota(jnp.int32, L)

  def zero_vec(i, _):
    zbuf[pl.ds(i * L, L)] = jnp.zeros((L,), jnp.float32)
    return 0
  lax.fori_loop(0, _ZP // L, zero_vec, 0)
  pltpu.sync_copy(zbuf, pacc.at[pl.ds(s * _ZP, _ZP)])
  plsc.subcore_barrier()

  tile_base = (c * NS + s) * _ROWS_PER_TILE

  def win(w, _):
    rbase = tile_base + w * _POOL_W
    pltpu.sync_copy(lg_hbm.at[pl.ds(rbase * LG, _POOL_W * LG)], rbuf)
    pltpu.sync_copy(batch_hbm.at[pl.ds(rbase, _POOL_W)], bbuf)

    mall = lane >= 0
    def expand(g, _):
      bv = bbuf[pl.ds(g * L, L)] * LG
      ppos = g * (L * LG) + lane * LG
      for j in range(LG):
        plsc.store_scatter(ibuf, [ppos + j], bv + j, mask=mall)
      return 0
    lax.fori_loop(0, _POOL_W // L, expand, 0)
    pltpu.sync_copy(rbuf, pacc.at[ibuf], add=True)
    return 0
  lax.fori_loop(0, _ROWS_PER_TILE // _POOL_W, win, 0)
  plsc.subcore_barrier()

  pltpu.sync_copy(pacc.at[pl.ds(s * _ZP, _ZP)],
                  out_hbm.at[c, pl.ds(s * _ZP, _ZP)])


def _k_pool(lg_flat, batch):
  f = pl.kernel(
      _pool_body,
      out_type=jax.ShapeDtypeStruct((NC, _PACC), jnp.float32),
      mesh=plsc.VectorSubcoreMesh(**_MESH),
      compiler_params=pltpu.CompilerParams(needs_layout_passes=False),
      scratch_types=[
          pltpu.VMEM_SHARED((_PACC,), jnp.float32),
          pltpu.VMEM((_ZP,), jnp.float32),
          pltpu.VMEM((_POOL_W * LG,), jnp.float32),
          pltpu.VMEM((_POOL_W,), jnp.int32),
          pltpu.VMEM((_POOL_W * LG,), jnp.int32),
      ],
  )
  return f(lg_flat, batch)


_PRE_B = 2048
_NR = N_NODES // NUM_IN
_PRE_R = _PRE_B // NUM_IN


def _pre_body(degp_ref, x_ref, dinv_ref, dinv2_ref, y0_ref):
  deg = 1.0 + degp_ref[0] + degp_ref[1]
  dinv = lax.rsqrt(deg)
  dinv_ref[...] = dinv
  dinv2_ref[...] = 1.0 / deg
  y0_ref[...] = x_ref[...] * dinv[:, :, None]


def _k_pre(degp, x):
  grid = N_NODES // _PRE_B
  return pl.pallas_call(
      _pre_body,
      grid=(grid,),
      in_specs=[
          pl.BlockSpec((NC, _PRE_R, NUM_IN), lambda i: (0, i, 0)),
          pl.BlockSpec((_PRE_R, NUM_IN, NUM_IN), lambda i: (i, 0, 0)),
      ],
      out_specs=[
          pl.BlockSpec((_PRE_R, NUM_IN), lambda i: (i, 0)),
          pl.BlockSpec((_PRE_R, NUM_IN), lambda i: (i, 0)),
          pl.BlockSpec((_PRE_R, NUM_IN, NUM_IN), lambda i: (i, 0, 0)),
      ],
      out_shape=[
          jax.ShapeDtypeStruct((_NR, NUM_IN), jnp.float32),
          jax.ShapeDtypeStruct((_NR, NUM_IN), jnp.float32),
          jax.ShapeDtypeStruct((_NR, NUM_IN, NUM_IN), jnp.float32),
      ],
  )(degp.reshape(NC, _NR, NUM_IN), x.reshape(_NR, NUM_IN, NUM_IN))


def _scale_body(y_ref, d_ref, o_ref):
  o_ref[...] = y_ref[...] * d_ref[...][:, :, None]


def _k_scale(y3d, d2d):
  grid = _NR // _PRE_R
  return pl.pallas_call(
      _scale_body,
      grid=(grid,),
      in_specs=[
          pl.BlockSpec((_PRE_R, NUM_IN, NUM_IN), lambda i: (i, 0, 0)),
          pl.BlockSpec((_PRE_R, NUM_IN), lambda i: (i, 0)),
      ],
      out_specs=pl.BlockSpec((_PRE_R, NUM_IN, NUM_IN), lambda i: (i, 0, 0)),
      out_shape=jax.ShapeDtypeStruct((_NR, NUM_IN, NUM_IN), jnp.float32),
  )(y3d, d2d)


_MM_B = 2048
_MM_R = _MM_B // NUM_IN


def _mm_body(y3_ref, dinv_ref, linwT_ref, linb_ref, fcwT_ref, o_ref):
  h2 = (y3_ref[...] * dinv_ref[...][:, :, None]).reshape(_MM_B, NUM_IN)
  h = jnp.dot(h2, linwT_ref[...], preferred_element_type=jnp.float32)
  h = jnp.maximum(h + linb_ref[...], 0.0)
  o_ref[...] = jnp.dot(h, fcwT_ref[...], preferred_element_type=jnp.float32)


def _k_mm(y3_3d, dinv2d, linwT, linb, fcwT):
  grid = N_NODES // _MM_B
  return pl.pallas_call(
      _mm_body,
      grid=(grid,),
      in_specs=[
          pl.BlockSpec((_MM_R, NUM_IN, NUM_IN), lambda i: (i, 0, 0)),
          pl.BlockSpec((_MM_R, NUM_IN), lambda i: (i, 0)),
          pl.BlockSpec((NUM_IN, NUM_HIDDEN), lambda i: (0, 0)),
          pl.BlockSpec((1, NUM_HIDDEN), lambda i: (0, 0)),
          pl.BlockSpec((NUM_HIDDEN, LG), lambda i: (0, 0)),
      ],
      out_specs=pl.BlockSpec((_MM_B, LG), lambda i: (i, 0)),
      out_shape=jax.ShapeDtypeStruct((N_NODES, LG), jnp.float32),
  )(y3_3d, dinv2d, linwT, linb, fcwT)


def _soft_body(pp_ref, fcb_ref, o_ref):
  z = pp_ref[0] + pp_ref[1] + fcb_ref[...]
  col = lax.broadcasted_iota(jnp.int32, (N_GRAPHS, LG), 1)
  valid = col < NUM_CLASS
  z = jnp.where(valid, z, -1e30)
  z = z - jnp.max(z, axis=1, keepdims=True)
  p = jnp.exp(z)
  p = jnp.where(valid, p, 0.0)
  o_ref[...] = p / jnp.sum(p, axis=1, keepdims=True)


def _k_soft(pooledp, fcb):
  return pl.pallas_call(
      _soft_body,
      in_specs=[
          pl.BlockSpec((NC, N_GRAPHS, LG), lambda: (0, 0, 0)),
          pl.BlockSpec((1, LG), lambda: (0, 0)),
      ],
      out_specs=pl.BlockSpec((N_GRAPHS, LG), lambda: (0, 0)),
      out_shape=jax.ShapeDtypeStruct((N_GRAPHS, LG), jnp.float32),
  )(pooledp, fcb)


def kernel(x, index, batch, weight, lin_w, lin_b, fc_w, fc_b):
  index = index.astype(jnp.int32)
  batch = batch.astype(jnp.int32)
  ew_win = jnp.tile(weight, _DEG_W // E_PER_GRAPH)
  w64pad = jnp.zeros((_WPAD + 16,), jnp.float32).at[:E_PER_GRAPH].set(weight)

  degp = _k_deg(index, ew_win)
  dinv, dinv2, y0_3d = _k_pre(degp, x)
  y1 = _k_hop(y0_3d.reshape(N_NODES, NUM_IN), index, w64pad)
  y2_3d = _k_scale(y1.reshape(_NR, NUM_IN, NUM_IN), dinv2)
  y3 = _k_hop(y2_3d.reshape(N_NODES, NUM_IN), index, w64pad)

  linwT = lin_w.T
  linb = lin_b.reshape(1, NUM_HIDDEN)
  fcwT = jnp.zeros((NUM_HIDDEN, LG), jnp.float32).at[:, :NUM_CLASS].set(fc_w.T)
  lg = _k_mm(y3.reshape(_NR, NUM_IN, NUM_IN), dinv, linwT, linb, fcwT)

  pooledp = _k_pool(lg.reshape(-1), batch)
  fcb = jnp.zeros((1, LG), jnp.float32).at[0, :NUM_CLASS].set(fc_b)
  probs = _k_soft(pooledp.reshape(NC, N_GRAPHS, LG), fcb)
  return probs[:, :NUM_CLASS]


# speedup vs baseline: 1.1042x; 1.1042x over previous
import functools

import jax
import jax.numpy as jnp
from jax import lax
from jax.experimental import pallas as pl
from jax.experimental.pallas import tpu as pltpu
import jax.experimental.pallas.tpu_sc as plsc

N_NODES = 262144
N_EDGES = 524288
NUM_IN = 128
NUM_HIDDEN = 256
NUM_CLASS = 10
N_GRAPHS = 8192
E_PER_GRAPH = 64

NC = 2
NS = 16
L = 16

_MESH = dict(core_axis_name="c", subcore_axis_name="s", num_cores=NC,
             num_subcores=NS)

_DEG_W = 4096
_E_PER_TILE_DEG = N_EDGES // (NC * NS)
_N_PER_TILE = N_NODES // NS


def _deg_body(index_hbm, ew_win_hbm, degp_hbm, dacc, zbuf, colbuf, ewb):
  c = lax.axis_index("c")
  s = lax.axis_index("s")

  def zero_vec(i, _):
    zbuf[pl.ds(i * L, L)] = jnp.zeros((L,), jnp.float32)
    return 0
  lax.fori_loop(0, _DEG_W // L, zero_vec, 0)

  def zero_chunk(i, _):
    pltpu.sync_copy(zbuf, dacc.at[pl.ds(s * _N_PER_TILE + i * _DEG_W, _DEG_W)])
    return 0
  lax.fori_loop(0, _N_PER_TILE // _DEG_W, zero_chunk, 0)
  plsc.subcore_barrier()

  pltpu.sync_copy(ew_win_hbm, ewb)
  tile_base = (c * NS + s) * _E_PER_TILE_DEG

  def win(w, _):
    ebase = tile_base + w * _DEG_W
    pltpu.sync_copy(index_hbm.at[1, pl.ds(ebase, _DEG_W)], colbuf)
    pltpu.sync_copy(ewb, dacc.at[colbuf], add=True)
    return 0
  lax.fori_loop(0, _E_PER_TILE_DEG // _DEG_W, win, 0)
  plsc.subcore_barrier()

  pltpu.sync_copy(dacc.at[pl.ds(s * _N_PER_TILE, _N_PER_TILE)],
                  degp_hbm.at[c, pl.ds(s * _N_PER_TILE, _N_PER_TILE)])


def _k_deg(index, ew_win):
  f = pl.kernel(
      _deg_body,
      out_type=jax.ShapeDtypeStruct((NC, N_NODES), jnp.float32),
      mesh=plsc.VectorSubcoreMesh(**_MESH),
      compiler_params=pltpu.CompilerParams(needs_layout_passes=False),
      scratch_types=[
          pltpu.VMEM_SHARED((N_NODES,), jnp.float32),
          pltpu.VMEM((_DEG_W,), jnp.float32),
          pltpu.VMEM((_DEG_W,), jnp.int32),
          pltpu.VMEM((_DEG_W,), jnp.float32),
      ],
  )
  return f(index, ew_win)


_R = 10240
_CHUNKS = 26
_PASSES = _CHUNKS // NC
_HOP_W = 2048
_UNROLL = 4
_E_PER_TILE = N_EDGES // NS
_GK = 128
_LCAP = 2208
_RPT = _R // NS
_TAIL_BASE = (_CHUNKS - 1) * _R
_TAIL_ROWS = N_NODES - _TAIL_BASE
_TAIL_RPT = _TAIL_ROWS // NS
_WPAD = 64
_LCBITS = 14
_LCMASK = (1 << _LCBITS) - 1


def _hop_finish(src_hbm, acc, st_r2, st_c2, st_w2, gbuf2, gsem, b):
  pltpu.make_async_copy(src_hbm.at[st_r2.at[b]], gbuf2.at[b], gsem).wait()

  def scale(g, _):
    wv = st_w2[b, pl.ds(g * L, L)]
    for k in range(L):
      wsp = jnp.full((L,), wv[k], jnp.float32)
      r = g * L + k
      for q in range(NUM_IN // L):
        gbuf2[b, r, pl.ds(q * L, L)] = gbuf2[b, r, pl.ds(q * L, L)] * wsp
    return 0
  lax.fori_loop(0, _GK // L, scale, 0)

  pltpu.sync_copy(gbuf2.at[b], acc.at[st_c2.at[b]], add=True)


def _hop_fire(src_hbm, acc, flr, flc, st_r2, st_c2, st_w2, gbuf2, ewb,
              gsem, start, fctr, mall):
  par = lax.rem(fctr, 2)

  def stage(i, _):
    st_r2[par, pl.ds(i * L, L)] = flr[pl.ds(start + i * L, L)]
    pk = flc[pl.ds(start + i * L, L)]
    st_c2[par, pl.ds(i * L, L)] = pk & _LCMASK
    st_w2[par, pl.ds(i * L, L)] = plsc.load_gather(
        ewb, [lax.shift_right_logical(pk, _LCBITS)], mask=mall)
    return 0
  lax.fori_loop(0, _GK // L, stage, 0)

  pltpu.async_copy(src_hbm.at[st_r2.at[par]], gbuf2.at[par], gsem).wait()

  def scale(g, _):
    wv = st_w2[par, pl.ds(g * L, L)]
    for k in range(L):
      wsp = jnp.full((L,), wv[k], jnp.float32)
      r = g * L + k
      for q in range(NUM_IN // L):
        gbuf2[par, r, pl.ds(q * L, L)] = gbuf2[par, r, pl.ds(q * L, L)] * wsp
    return 0
  lax.fori_loop(0, _GK // L, scale, 0)

  pltpu.sync_copy(gbuf2.at[par], acc.at[st_c2.at[par]], add=True)


def _hop_body(src_hbm, index_hbm, w64_hbm, dst_hbm,
              acc, flr, flc, st_r2, st_c2, st_w2, gbuf2, colw, roww, ewb,
              gsem, wsem):
  c = lax.axis_index("c")
  s = lax.axis_index("s")
  pltpu.sync_copy(w64_hbm, ewb)
  lane = lax.iota(jnp.int32, L)
  mall = lane >= 0
  tile_e = s * _E_PER_TILE
  n_win = _E_PER_TILE // _HOP_W

  def do_pass(p, _):
    k = NC * p + c
    is_tail = k == (_CHUNKS - 1)
    base = jnp.where(is_tail, _TAIL_BASE, k * _R)
    init_base = jnp.where(is_tail, N_NODES - _R, k * _R)
    hi = jnp.where(is_tail, N_NODES, k * _R + _R)

    pltpu.sync_copy(src_hbm.at[pl.ds(init_base + s * _RPT, _RPT)],
                    acc.at[pl.ds(s * _RPT, _RPT)])
    plsc.subcore_barrier()

    def drain(cnt, fired, fctr):
      def one(d, carry):
        fired, fctr = carry
        go = fired + _GK <= cnt
        @pl.when(go)
        def _():
          _hop_fire(src_hbm, acc, flr, flc, st_r2, st_c2, st_w2, gbuf2,
                    ewb, gsem, fired, fctr, mall)
        adv = jnp.where(go, 1, 0)
        return (fired + adv * _GK, fctr + adv)
      fired, fctr = lax.fori_loop(0, (_LCAP + _GK - 1) // _GK, one,
                                  (fired, fctr))
      rem = cnt - fired
      def mv(i, _):
        @pl.when(i * L < rem)
        def _():
          flr[pl.ds(i * L, L)] = flr[pl.ds(fired + i * L, L)]
          flc[pl.ds(i * L, L)] = flc[pl.ds(fired + i * L, L)]
        return 0
      lax.fori_loop(0, _GK // L, mv, 0)
      return rem, fctr

    def win(w, carry):
      cnt, fctr = carry
      ebase = tile_e + w * _HOP_W
      pltpu.async_copy(index_hbm.at[1, pl.ds(ebase, _HOP_W)], colw, wsem)
      pltpu.async_copy(index_hbm.at[0, pl.ds(ebase, _HOP_W)], roww, wsem)
      pltpu.make_async_copy(index_hbm.at[1, pl.ds(ebase, _HOP_W)], colw,
                            wsem).wait()
      pltpu.make_async_copy(index_hbm.at[0, pl.ds(ebase, _HOP_W)], roww,
                            wsem).wait()

      def scan(jj, cnt):
        for u in range(_UNROLL):
          j = jj * _UNROLL + u
          c16 = colw[pl.ds(j * L, L)]
          m = (c16 >= base) & (c16 < hi)
          r16 = roww[pl.ds(j * L, L)]
          widx16 = lax.rem(j, 4) * L + lane
          pk16 = (c16 - init_base) | lax.shift_left(widx16, _LCBITS)
          cs = plsc.cumsum(m.astype(jnp.int32))
          pos = cs + (cnt - 1)
          plsc.store_scatter(flr, [pos], r16, mask=m)
          plsc.store_scatter(flc, [pos], pk16, mask=m)
          cnt = cnt + cs[L - 1]
        return cnt
      cnt = lax.fori_loop(0, _HOP_W // L // _UNROLL, scan, cnt)
      return drain(cnt, 0, fctr)

    cnt, fctr = lax.fori_loop(0, n_win, win, (0, 0))

    cnt_pad = jnp.where(cnt > 0, ((cnt + _GK - 1) // _GK) * _GK, 0)
    pad_pk = lane | (_WPAD << _LCBITS)
    def pad(i, _):
      @pl.when(cnt + i * L < cnt_pad)
      def _():
        off = cnt + i * L
        flr[pl.ds(off, L)] = lane
        flc[pl.ds(off, L)] = pad_pk
      return 0
    lax.fori_loop(0, _GK // L, pad, 0)
    _, fctr = drain(cnt_pad, 0, fctr)

    plsc.subcore_barrier()
    @pl.when(jnp.logical_not(is_tail))
    def _():
      pltpu.sync_copy(acc.at[pl.ds(s * _RPT, _RPT)],
                      dst_hbm.at[pl.ds(init_base + s * _RPT, _RPT)])
    @pl.when(is_tail)
    def _():
      pltpu.sync_copy(
          acc.at[pl.ds(_R - _TAIL_ROWS + s * _TAIL_RPT, _TAIL_RPT)],
          dst_hbm.at[pl.ds(_TAIL_BASE + s * _TAIL_RPT, _TAIL_RPT)])
    plsc.subcore_barrier()
    return 0

  lax.fori_loop(0, _PASSES, do_pass, 0)


def _k_hop(src, index, w64pad):
  f = pl.kernel(
      _hop_body,
      out_type=jax.ShapeDtypeStruct((N_NODES, NUM_IN), jnp.float32),
      mesh=plsc.VectorSubcoreMesh(**_MESH),
      compiler_params=pltpu.CompilerParams(needs_layout_passes=False),
      scratch_types=[
          pltpu.VMEM_SHARED((_R, NUM_IN), jnp.float32),
          pltpu.VMEM((_LCAP,), jnp.int32),
          pltpu.VMEM((_LCAP,), jnp.int32),
          pltpu.VMEM((2, _GK), jnp.int32),
          pltpu.VMEM((2, _GK), jnp.int32),
          pltpu.VMEM((2, _GK), jnp.float32),
          pltpu.VMEM((2, _GK, NUM_IN), jnp.float32),
          pltpu.VMEM((_HOP_W,), jnp.int32),
          pltpu.VMEM((_HOP_W,), jnp.int32),
          pltpu.VMEM((_WPAD + 16,), jnp.float32),
          pltpu.SemaphoreType.DMA,
          pltpu.SemaphoreType.DMA,
      ],
  )
  return f(src, index, w64pad)


_POOL_W = 2048
_ROWS_PER_TILE = N_NODES // (NC * NS)
_G_PER_TILE = N_GRAPHS // NS
LG = 16


_PACC = N_GRAPHS * LG
_ZP = _PACC // NS


def _pool_body(lg_hbm, batch_hbm, out_hbm, pacc, zbuf, rbuf, bbuf, ibuf):
  c = lax.axis_index("c")
  s = lax.axis_index("s")
  lane = lax.iota(jnp.int32, L)

  def zero_vec(i, _):
    zbuf[pl.ds(i * L, L)] = jnp.zeros((L,), jnp.float32)
    return 0
  lax.fori_loop(0, _ZP // L, zero_vec, 0)
  pltpu.sync_copy(zbuf, pacc.at[pl.ds(s * _ZP, _ZP)])
  plsc.subcore_barrier()

  tile_base = (c * NS + s) * _ROWS_PER_TILE

  def win(w, _):
    rbase = tile_base + w * _POOL_W
    pltpu.sync_copy(lg_hbm.at[pl.ds(rbase * LG, _POOL_W * LG)], rbuf)
    pltpu.sync_copy(batch_hbm.at[pl.ds(rbase, _POOL_W)], bbuf)

    mall = lane >= 0
    def expand(g, _):
      bv = bbuf[pl.ds(g * L, L)] * LG
      ppos = g * (L * LG) + lane * LG
      for j in range(LG):
        plsc.store_scatter(ibuf, [ppos + j], bv + j, mask=mall)
      return 0
    lax.fori_loop(0, _POOL_W // L, expand, 0)
    pltpu.sync_copy(rbuf, pacc.at[ibuf], add=True)
    return 0
  lax.fori_loop(0, _ROWS_PER_TILE // _POOL_W, win, 0)
  plsc.subcore_barrier()

  pltpu.sync_copy(pacc.at[pl.ds(s * _ZP, _ZP)],
                  out_hbm.at[c, pl.ds(s * _ZP, _ZP)])


def _k_pool(lg_flat, batch):
  f = pl.kernel(
      _pool_body,
      out_type=jax.ShapeDtypeStruct((NC, _PACC), jnp.float32),
      mesh=plsc.VectorSubcoreMesh(**_MESH),
      compiler_params=pltpu.CompilerParams(needs_layout_passes=False),
      scratch_types=[
          pltpu.VMEM_SHARED((_PACC,), jnp.float32),
          pltpu.VMEM((_ZP,), jnp.float32),
          pltpu.VMEM((_POOL_W * LG,), jnp.float32),
          pltpu.VMEM((_POOL_W,), jnp.int32),
          pltpu.VMEM((_POOL_W * LG,), jnp.int32),
      ],
  )
  return f(lg_flat, batch)


_PRE_B = 2048
_NR = N_NODES // NUM_IN
_PRE_R = _PRE_B // NUM_IN


def _pre_body(degp_ref, x_ref, dinv_ref, dinv2_ref, y0_ref):
  deg = 1.0 + degp_ref[0] + degp_ref[1]
  dinv = lax.rsqrt(deg)
  dinv_ref[...] = dinv
  dinv2_ref[...] = 1.0 / deg
  y0_ref[...] = x_ref[...] * dinv[:, :, None]


def _k_pre(degp, x):
  grid = N_NODES // _PRE_B
  return pl.pallas_call(
      _pre_body,
      grid=(grid,),
      in_specs=[
          pl.BlockSpec((NC, _PRE_R, NUM_IN), lambda i: (0, i, 0)),
          pl.BlockSpec((_PRE_R, NUM_IN, NUM_IN), lambda i: (i, 0, 0)),
      ],
      out_specs=[
          pl.BlockSpec((_PRE_R, NUM_IN), lambda i: (i, 0)),
          pl.BlockSpec((_PRE_R, NUM_IN), lambda i: (i, 0)),
          pl.BlockSpec((_PRE_R, NUM_IN, NUM_IN), lambda i: (i, 0, 0)),
      ],
      out_shape=[
          jax.ShapeDtypeStruct((_NR, NUM_IN), jnp.float32),
          jax.ShapeDtypeStruct((_NR, NUM_IN), jnp.float32),
          jax.ShapeDtypeStruct((_NR, NUM_IN, NUM_IN), jnp.float32),
      ],
  )(degp.reshape(NC, _NR, NUM_IN), x.reshape(_NR, NUM_IN, NUM_IN))


def _scale_body(y_ref, d_ref, o_ref):
  o_ref[...] = y_ref[...] * d_ref[...][:, :, None]


def _k_scale(y3d, d2d):
  grid = _NR // _PRE_R
  return pl.pallas_call(
      _scale_body,
      grid=(grid,),
      in_specs=[
          pl.BlockSpec((_PRE_R, NUM_IN, NUM_IN), lambda i: (i, 0, 0)),
          pl.BlockSpec((_PRE_R, NUM_IN), lambda i: (i, 0)),
      ],
      out_specs=pl.BlockSpec((_PRE_R, NUM_IN, NUM_IN), lambda i: (i, 0, 0)),
      out_shape=jax.ShapeDtypeStruct((_NR, NUM_IN, NUM_IN), jnp.float32),
  )(y3d, d2d)


_MM_B = 2048
_MM_R = _MM_B // NUM_IN


def _mm_body(y3_ref, dinv_ref, linwT_ref, linb_ref, fcwT_ref, o_ref):
  h2 = (y3_ref[...] * dinv_ref[...][:, :, None]).reshape(_MM_B, NUM_IN)
  h = jnp.dot(h2, linwT_ref[...], preferred_element_type=jnp.float32)
  h = jnp.maximum(h + linb_ref[...], 0.0)
  o_ref[...] = jnp.dot(h, fcwT_ref[...], preferred_element_type=jnp.float32)


def _k_mm(y3_3d, dinv2d, linwT, linb, fcwT):
  grid = N_NODES // _MM_B
  return pl.pallas_call(
      _mm_body,
      grid=(grid,),
      in_specs=[
          pl.BlockSpec((_MM_R, NUM_IN, NUM_IN), lambda i: (i, 0, 0)),
          pl.BlockSpec((_MM_R, NUM_IN), lambda i: (i, 0)),
          pl.BlockSpec((NUM_IN, NUM_HIDDEN), lambda i: (0, 0)),
          pl.BlockSpec((1, NUM_HIDDEN), lambda i: (0, 0)),
          pl.BlockSpec((NUM_HIDDEN, LG), lambda i: (0, 0)),
      ],
      out_specs=pl.BlockSpec((_MM_B, LG), lambda i: (i, 0)),
      out_shape=jax.ShapeDtypeStruct((N_NODES, LG), jnp.float32),
  )(y3_3d, dinv2d, linwT, linb, fcwT)


def _soft_body(pp_ref, fcb_ref, o_ref):
  z = pp_ref[0] + pp_ref[1] + fcb_ref[...]
  col = lax.broadcasted_iota(jnp.int32, (N_GRAPHS, LG), 1)
  valid = col < NUM_CLASS
  z = jnp.where(valid, z, -1e30)
  z = z - jnp.max(z, axis=1, keepdims=True)
  p = jnp.exp(z)
  p = jnp.where(valid, p, 0.0)
  o_ref[...] = p / jnp.sum(p, axis=1, keepdims=True)


def _k_soft(pooledp, fcb):
  return pl.pallas_call(
      _soft_body,
      in_specs=[
          pl.BlockSpec((NC, N_GRAPHS, LG), lambda: (0, 0, 0)),
          pl.BlockSpec((1, LG), lambda: (0, 0)),
      ],
      out_specs=pl.BlockSpec((N_GRAPHS, LG), lambda: (0, 0)),
      out_shape=jax.ShapeDtypeStruct((N_GRAPHS, LG), jnp.float32),
  )(pooledp, fcb)


def kernel(x, index, batch, weight, lin_w, lin_b, fc_w, fc_b):
  index = index.astype(jnp.int32)
  batch = batch.astype(jnp.int32)
  ew_win = jnp.tile(weight, _DEG_W // E_PER_GRAPH)
  w64pad = jnp.zeros((_WPAD + 16,), jnp.float32).at[:E_PER_GRAPH].set(weight)

  degp = _k_deg(index, ew_win)
  dinv, dinv2, y0_3d = _k_pre(degp, x)
  y1 = _k_hop(y0_3d.reshape(N_NODES, NUM_IN), index, w64pad)
  y2_3d = _k_scale(y1.reshape(_NR, NUM_IN, NUM_IN), dinv2)
  y3 = _k_hop(y2_3d.reshape(N_NODES, NUM_IN), index, w64pad)

  linwT = lin_w.T
  linb = lin_b.reshape(1, NUM_HIDDEN)
  fcwT = jnp.zeros((NUM_HIDDEN, LG), jnp.float32).at[:, :NUM_CLASS].set(fc_w.T)
  lg = _k_mm(y3.reshape(_NR, NUM_IN, NUM_IN), dinv, linwT, linb, fcwT)

  pooledp = _k_pool(lg.reshape(-1), batch)
  fcb = jnp.zeros((1, LG), jnp.float32).at[0, :NUM_CLASS].set(fc_b)
  probs = _k_soft(pooledp.reshape(NC, N_GRAPHS, LG), fcb)
  return probs[:, :NUM_CLASS]


# speedup vs baseline: 1.1059x; 1.0016x over previous
import functools

import jax
import jax.numpy as jnp
from jax import lax
from jax.experimental import pallas as pl
from jax.experimental.pallas import tpu as pltpu
import jax.experimental.pallas.tpu_sc as plsc

N_NODES = 262144
N_EDGES = 524288
NUM_IN = 128
NUM_HIDDEN = 256
NUM_CLASS = 10
N_GRAPHS = 8192
E_PER_GRAPH = 64

NC = 2
NS = 16
L = 16

_MESH = dict(core_axis_name="c", subcore_axis_name="s", num_cores=NC,
             num_subcores=NS)

_DEG_W = 4096
_E_PER_TILE_DEG = N_EDGES // (NC * NS)
_N_PER_TILE = N_NODES // NS


def _deg_body(index_hbm, ew_win_hbm, degp_hbm, dacc, zbuf, colbuf, ewb):
  c = lax.axis_index("c")
  s = lax.axis_index("s")

  def zero_vec(i, _):
    zbuf[pl.ds(i * L, L)] = jnp.zeros((L,), jnp.float32)
    return 0
  lax.fori_loop(0, _DEG_W // L, zero_vec, 0)

  def zero_chunk(i, _):
    pltpu.sync_copy(zbuf, dacc.at[pl.ds(s * _N_PER_TILE + i * _DEG_W, _DEG_W)])
    return 0
  lax.fori_loop(0, _N_PER_TILE // _DEG_W, zero_chunk, 0)
  plsc.subcore_barrier()

  pltpu.sync_copy(ew_win_hbm, ewb)
  tile_base = (c * NS + s) * _E_PER_TILE_DEG

  def win(w, _):
    ebase = tile_base + w * _DEG_W
    pltpu.sync_copy(index_hbm.at[1, pl.ds(ebase, _DEG_W)], colbuf)
    pltpu.sync_copy(ewb, dacc.at[colbuf], add=True)
    return 0
  lax.fori_loop(0, _E_PER_TILE_DEG // _DEG_W, win, 0)
  plsc.subcore_barrier()

  pltpu.sync_copy(dacc.at[pl.ds(s * _N_PER_TILE, _N_PER_TILE)],
                  degp_hbm.at[c, pl.ds(s * _N_PER_TILE, _N_PER_TILE)])


def _k_deg(index, ew_win):
  f = pl.kernel(
      _deg_body,
      out_type=jax.ShapeDtypeStruct((NC, N_NODES), jnp.float32),
      mesh=plsc.VectorSubcoreMesh(**_MESH),
      compiler_params=pltpu.CompilerParams(needs_layout_passes=False),
      scratch_types=[
          pltpu.VMEM_SHARED((N_NODES,), jnp.float32),
          pltpu.VMEM((_DEG_W,), jnp.float32),
          pltpu.VMEM((_DEG_W,), jnp.int32),
          pltpu.VMEM((_DEG_W,), jnp.float32),
      ],
  )
  return f(index, ew_win)


_R = 10240
_CHUNKS = 26
_PASSES = _CHUNKS // NC
_HOP_W = 2048
_UNROLL = 1
_E_PER_TILE = N_EDGES // NS
_GK = 128
_LCAP = 2208
_RPT = _R // NS
_TAIL_BASE = (_CHUNKS - 1) * _R
_TAIL_ROWS = N_NODES - _TAIL_BASE
_TAIL_RPT = _TAIL_ROWS // NS
_WPAD = 64
_LCBITS = 14
_LCMASK = (1 << _LCBITS) - 1


def _hop_finish(src_hbm, acc, st_r2, st_c2, st_w2, gbuf2, gsem, b):
  pltpu.make_async_copy(src_hbm.at[st_r2.at[b]], gbuf2.at[b], gsem).wait()

  def scale(g, _):
    wv = st_w2[b, pl.ds(g * L, L)]
    for k in range(L):
      wsp = jnp.full((L,), wv[k], jnp.float32)
      r = g * L + k
      for q in range(NUM_IN // L):
        gbuf2[b, r, pl.ds(q * L, L)] = gbuf2[b, r, pl.ds(q * L, L)] * wsp
    return 0
  lax.fori_loop(0, _GK // L, scale, 0)

  pltpu.sync_copy(gbuf2.at[b], acc.at[st_c2.at[b]], add=True)


def _hop_fire(src_hbm, acc, flr, flc, st_r2, st_c2, st_w2, gbuf2, ewb,
              gsem, start, fctr, mall):
  par = lax.rem(fctr, 2)

  def stage(i, _):
    st_r2[par, pl.ds(i * L, L)] = flr[pl.ds(start + i * L, L)]
    pk = flc[pl.ds(start + i * L, L)]
    st_c2[par, pl.ds(i * L, L)] = pk & _LCMASK
    st_w2[par, pl.ds(i * L, L)] = plsc.load_gather(
        ewb, [lax.shift_right_logical(pk, _LCBITS)], mask=mall)
    return 0
  lax.fori_loop(0, _GK // L, stage, 0)

  pltpu.async_copy(src_hbm.at[st_r2.at[par]], gbuf2.at[par], gsem).wait()

  def scale(g, _):
    wv = st_w2[par, pl.ds(g * L, L)]
    for k in range(L):
      wsp = jnp.full((L,), wv[k], jnp.float32)
      r = g * L + k
      for q in range(NUM_IN // L):
        gbuf2[par, r, pl.ds(q * L, L)] = gbuf2[par, r, pl.ds(q * L, L)] * wsp
    return 0
  lax.fori_loop(0, _GK // L, scale, 0)

  pltpu.sync_copy(gbuf2.at[par], acc.at[st_c2.at[par]], add=True)


def _hop_body(src_hbm, index_hbm, w64_hbm, dst_hbm,
              acc, flr, flc, st_r2, st_c2, st_w2, gbuf2, colw, roww, ewb,
              gsem, wsem):
  c = lax.axis_index("c")
  s = lax.axis_index("s")
  pltpu.sync_copy(w64_hbm, ewb)
  lane = lax.iota(jnp.int32, L)
  mall = lane >= 0
  tile_e = s * _E_PER_TILE
  n_win = _E_PER_TILE // _HOP_W

  def do_pass(p, _):
    k = NC * p + c
    is_tail = k == (_CHUNKS - 1)
    base = jnp.where(is_tail, _TAIL_BASE, k * _R)
    init_base = jnp.where(is_tail, N_NODES - _R, k * _R)
    hi = jnp.where(is_tail, N_NODES, k * _R + _R)

    pltpu.sync_copy(src_hbm.at[pl.ds(init_base + s * _RPT, _RPT)],
                    acc.at[pl.ds(s * _RPT, _RPT)])
    plsc.subcore_barrier()

    def drain(cnt, fired, fctr):
      def one(d, carry):
        fired, fctr = carry
        go = fired + _GK <= cnt
        @pl.when(go)
        def _():
          _hop_fire(src_hbm, acc, flr, flc, st_r2, st_c2, st_w2, gbuf2,
                    ewb, gsem, fired, fctr, mall)
        adv = jnp.where(go, 1, 0)
        return (fired + adv * _GK, fctr + adv)
      fired, fctr = lax.fori_loop(0, (_LCAP + _GK - 1) // _GK, one,
                                  (fired, fctr))
      rem = cnt - fired
      def mv(i, _):
        @pl.when(i * L < rem)
        def _():
          flr[pl.ds(i * L, L)] = flr[pl.ds(fired + i * L, L)]
          flc[pl.ds(i * L, L)] = flc[pl.ds(fired + i * L, L)]
        return 0
      lax.fori_loop(0, _GK // L, mv, 0)
      return rem, fctr

    def win(w, carry):
      cnt, fctr = carry
      ebase = tile_e + w * _HOP_W
      pltpu.async_copy(index_hbm.at[1, pl.ds(ebase, _HOP_W)], colw, wsem)
      pltpu.async_copy(index_hbm.at[0, pl.ds(ebase, _HOP_W)], roww, wsem)
      pltpu.make_async_copy(index_hbm.at[1, pl.ds(ebase, _HOP_W)], colw,
                            wsem).wait()
      pltpu.make_async_copy(index_hbm.at[0, pl.ds(ebase, _HOP_W)], roww,
                            wsem).wait()

      def scan(jj, cnt):
        for u in range(_UNROLL):
          j = jj * _UNROLL + u
          c16 = colw[pl.ds(j * L, L)]
          m = (c16 >= base) & (c16 < hi)
          r16 = roww[pl.ds(j * L, L)]
          widx16 = lax.rem(j, 4) * L + lane
          pk16 = (c16 - init_base) | lax.shift_left(widx16, _LCBITS)
          cs = plsc.cumsum(m.astype(jnp.int32))
          pos = cs + (cnt - 1)
          plsc.store_scatter(flr, [pos], r16, mask=m)
          plsc.store_scatter(flc, [pos], pk16, mask=m)
          cnt = cnt + cs[L - 1]
        return cnt
      cnt = lax.fori_loop(0, _HOP_W // L // _UNROLL, scan, cnt)
      return drain(cnt, 0, fctr)

    cnt, fctr = lax.fori_loop(0, n_win, win, (0, 0))

    cnt_pad = jnp.where(cnt > 0, ((cnt + _GK - 1) // _GK) * _GK, 0)
    pad_pk = lane | (_WPAD << _LCBITS)
    def pad(i, _):
      @pl.when(cnt + i * L < cnt_pad)
      def _():
        off = cnt + i * L
        flr[pl.ds(off, L)] = lane
        flc[pl.ds(off, L)] = pad_pk
      return 0
    lax.fori_loop(0, _GK // L, pad, 0)
    _, fctr = drain(cnt_pad, 0, fctr)

    plsc.subcore_barrier()
    @pl.when(jnp.logical_not(is_tail))
    def _():
      pltpu.sync_copy(acc.at[pl.ds(s * _RPT, _RPT)],
                      dst_hbm.at[pl.ds(init_base + s * _RPT, _RPT)])
    @pl.when(is_tail)
    def _():
      pltpu.sync_copy(
          acc.at[pl.ds(_R - _TAIL_ROWS + s * _TAIL_RPT, _TAIL_RPT)],
          dst_hbm.at[pl.ds(_TAIL_BASE + s * _TAIL_RPT, _TAIL_RPT)])
    plsc.subcore_barrier()
    return 0

  lax.fori_loop(0, _PASSES, do_pass, 0)


def _k_hop(src, index, w64pad):
  f = pl.kernel(
      _hop_body,
      out_type=jax.ShapeDtypeStruct((N_NODES, NUM_IN), jnp.float32),
      mesh=plsc.VectorSubcoreMesh(**_MESH),
      compiler_params=pltpu.CompilerParams(needs_layout_passes=False),
      scratch_types=[
          pltpu.VMEM_SHARED((_R, NUM_IN), jnp.float32),
          pltpu.VMEM((_LCAP,), jnp.int32),
          pltpu.VMEM((_LCAP,), jnp.int32),
          pltpu.VMEM((2, _GK), jnp.int32),
          pltpu.VMEM((2, _GK), jnp.int32),
          pltpu.VMEM((2, _GK), jnp.float32),
          pltpu.VMEM((2, _GK, NUM_IN), jnp.float32),
          pltpu.VMEM((_HOP_W,), jnp.int32),
          pltpu.VMEM((_HOP_W,), jnp.int32),
          pltpu.VMEM((_WPAD + 16,), jnp.float32),
          pltpu.SemaphoreType.DMA,
          pltpu.SemaphoreType.DMA,
      ],
  )
  return f(src, index, w64pad)


_POOL_W = 2048
_ROWS_PER_TILE = N_NODES // (NC * NS)
_G_PER_TILE = N_GRAPHS // NS
LG = 16


_PACC = N_GRAPHS * LG
_ZP = _PACC // NS


def _pool_body(lg_hbm, batch_hbm, out_hbm, pacc, zbuf, rbuf, bbuf, ibuf):
  c = lax.axis_index("c")
  s = lax.axis_index("s")
  lane = lax.iota(jnp.int32, L)

  def zero_vec(i, _):
    zbuf[pl.ds(i * L, L)] = jnp.zeros((L,), jnp.float32)
    return 0
  lax.fori_loop(0, _ZP // L, zero_vec, 0)
  pltpu.sync_copy(zbuf, pacc.at[pl.ds(s * _ZP, _ZP)])
  plsc.subcore_barrier()

  tile_base = (c * NS + s) * _ROWS_PER_TILE

  def win(w, _):
    rbase = tile_base + w * _POOL_W
    pltpu.sync_copy(lg_hbm.at[pl.ds(rbase * LG, _POOL_W * LG)], rbuf)
    pltpu.sync_copy(batch_hbm.at[pl.ds(rbase, _POOL_W)], bbuf)

    mall = lane >= 0
    def expand(g, _):
      bv = bbuf[pl.ds(g * L, L)] * LG
      ppos = g * (L * LG) + lane * LG
      for j in range(LG):
        plsc.store_scatter(ibuf, [ppos + j], bv + j, mask=mall)
      return 0
    lax.fori_loop(0, _POOL_W // L, expand, 0)
    pltpu.sync_copy(rbuf, pacc.at[ibuf], add=True)
    return 0
  lax.fori_loop(0, _ROWS_PER_TILE // _POOL_W, win, 0)
  plsc.subcore_barrier()

  pltpu.sync_copy(pacc.at[pl.ds(s * _ZP, _ZP)],
                  out_hbm.at[c, pl.ds(s * _ZP, _ZP)])


def _k_pool(lg_flat, batch):
  f = pl.kernel(
      _pool_body,
      out_type=jax.ShapeDtypeStruct((NC, _PACC), jnp.float32),
      mesh=plsc.VectorSubcoreMesh(**_MESH),
      compiler_params=pltpu.CompilerParams(needs_layout_passes=False),
      scratch_types=[
          pltpu.VMEM_SHARED((_PACC,), jnp.float32),
          pltpu.VMEM((_ZP,), jnp.float32),
          pltpu.VMEM((_POOL_W * LG,), jnp.float32),
          pltpu.VMEM((_POOL_W,), jnp.int32),
          pltpu.VMEM((_POOL_W * LG,), jnp.int32),
      ],
  )
  return f(lg_flat, batch)


_PRE_B = 2048
_NR = N_NODES // NUM_IN
_PRE_R = _PRE_B // NUM_IN


def _pre_body(degp_ref, x_ref, dinv_ref, dinv2_ref, y0_ref):
  deg = 1.0 + degp_ref[0] + degp_ref[1]
  dinv = lax.rsqrt(deg)
  dinv_ref[...] = dinv
  dinv2_ref[...] = 1.0 / deg
  y0_ref[...] = x_ref[...] * dinv[:, :, None]


def _k_pre(degp, x):
  grid = N_NODES // _PRE_B
  return pl.pallas_call(
      _pre_body,
      grid=(grid,),
      in_specs=[
          pl.BlockSpec((NC, _PRE_R, NUM_IN), lambda i: (0, i, 0)),
          pl.BlockSpec((_PRE_R, NUM_IN, NUM_IN), lambda i: (i, 0, 0)),
      ],
      out_specs=[
          pl.BlockSpec((_PRE_R, NUM_IN), lambda i: (i, 0)),
          pl.BlockSpec((_PRE_R, NUM_IN), lambda i: (i, 0)),
          pl.BlockSpec((_PRE_R, NUM_IN, NUM_IN), lambda i: (i, 0, 0)),
      ],
      out_shape=[
          jax.ShapeDtypeStruct((_NR, NUM_IN), jnp.float32),
          jax.ShapeDtypeStruct((_NR, NUM_IN), jnp.float32),
          jax.ShapeDtypeStruct((_NR, NUM_IN, NUM_IN), jnp.float32),
      ],
  )(degp.reshape(NC, _NR, NUM_IN), x.reshape(_NR, NUM_IN, NUM_IN))


def _scale_body(y_ref, d_ref, o_ref):
  o_ref[...] = y_ref[...] * d_ref[...][:, :, None]


def _k_scale(y3d, d2d):
  grid = _NR // _PRE_R
  return pl.pallas_call(
      _scale_body,
      grid=(grid,),
      in_specs=[
          pl.BlockSpec((_PRE_R, NUM_IN, NUM_IN), lambda i: (i, 0, 0)),
          pl.BlockSpec((_PRE_R, NUM_IN), lambda i: (i, 0)),
      ],
      out_specs=pl.BlockSpec((_PRE_R, NUM_IN, NUM_IN), lambda i: (i, 0, 0)),
      out_shape=jax.ShapeDtypeStruct((_NR, NUM_IN, NUM_IN), jnp.float32),
  )(y3d, d2d)


_MM_B = 2048
_MM_R = _MM_B // NUM_IN


def _mm_body(y3_ref, dinv_ref, linwT_ref, linb_ref, fcwT_ref, o_ref):
  h2 = (y3_ref[...] * dinv_ref[...][:, :, None]).reshape(_MM_B, NUM_IN)
  h = jnp.dot(h2, linwT_ref[...], preferred_element_type=jnp.float32)
  h = jnp.maximum(h + linb_ref[...], 0.0)
  o_ref[...] = jnp.dot(h, fcwT_ref[...], preferred_element_type=jnp.float32)


def _k_mm(y3_3d, dinv2d, linwT, linb, fcwT):
  grid = N_NODES // _MM_B
  return pl.pallas_call(
      _mm_body,
      grid=(grid,),
      in_specs=[
          pl.BlockSpec((_MM_R, NUM_IN, NUM_IN), lambda i: (i, 0, 0)),
          pl.BlockSpec((_MM_R, NUM_IN), lambda i: (i, 0)),
          pl.BlockSpec((NUM_IN, NUM_HIDDEN), lambda i: (0, 0)),
          pl.BlockSpec((1, NUM_HIDDEN), lambda i: (0, 0)),
          pl.BlockSpec((NUM_HIDDEN, LG), lambda i: (0, 0)),
      ],
      out_specs=pl.BlockSpec((_MM_B, LG), lambda i: (i, 0)),
      out_shape=jax.ShapeDtypeStruct((N_NODES, LG), jnp.float32),
  )(y3_3d, dinv2d, linwT, linb, fcwT)


def _soft_body(pp_ref, fcb_ref, o_ref):
  z = pp_ref[0] + pp_ref[1] + fcb_ref[...]
  col = lax.broadcasted_iota(jnp.int32, (N_GRAPHS, LG), 1)
  valid = col < NUM_CLASS
  z = jnp.where(valid, z, -1e30)
  z = z - jnp.max(z, axis=1, keepdims=True)
  p = jnp.exp(z)
  p = jnp.where(valid, p, 0.0)
  o_ref[...] = p / jnp.sum(p, axis=1, keepdims=True)


def _k_soft(pooledp, fcb):
  return pl.pallas_call(
      _soft_body,
      in_specs=[
          pl.BlockSpec((NC, N_GRAPHS, LG), lambda: (0, 0, 0)),
          pl.BlockSpec((1, LG), lambda: (0, 0)),
      ],
      out_specs=pl.BlockSpec((N_GRAPHS, LG), lambda: (0, 0)),
      out_shape=jax.ShapeDtypeStruct((N_GRAPHS, LG), jnp.float32),
  )(pooledp, fcb)


def kernel(x, index, batch, weight, lin_w, lin_b, fc_w, fc_b):
  index = index.astype(jnp.int32)
  batch = batch.astype(jnp.int32)
  ew_win = jnp.tile(weight, _DEG_W // E_PER_GRAPH)
  w64pad = jnp.zeros((_WPAD + 16,), jnp.float32).at[:E_PER_GRAPH].set(weight)

  degp = _k_deg(index, ew_win)
  dinv, dinv2, y0_3d = _k_pre(degp, x)
  y1 = _k_hop(y0_3d.reshape(N_NODES, NUM_IN), index, w64pad)
  y2_3d = _k_scale(y1.reshape(_NR, NUM_IN, NUM_IN), dinv2)
  y3 = _k_hop(y2_3d.reshape(N_NODES, NUM_IN), index, w64pad)

  linwT = lin_w.T
  linb = lin_b.reshape(1, NUM_HIDDEN)
  fcwT = jnp.zeros((NUM_HIDDEN, LG), jnp.float32).at[:, :NUM_CLASS].set(fc_w.T)
  lg = _k_mm(y3.reshape(_NR, NUM_IN, NUM_IN), dinv, linwT, linb, fcwT)

  pooledp = _k_pool(lg.reshape(-1), batch)
  fcb = jnp.zeros((1, LG), jnp.float32).at[0, :NUM_CLASS].set(fc_b)
  probs = _k_soft(pooledp.reshape(NC, N_GRAPHS, LG), fcb)
  return probs[:, :NUM_CLASS]


# speedup vs baseline: 1.5135x; 1.3686x over previous
import functools

import jax
import jax.numpy as jnp
from jax import lax
from jax.experimental import pallas as pl
from jax.experimental.pallas import tpu as pltpu
import jax.experimental.pallas.tpu_sc as plsc

N_NODES = 262144
N_EDGES = 524288
NUM_IN = 128
NUM_HIDDEN = 256
NUM_CLASS = 10
N_GRAPHS = 8192
E_PER_GRAPH = 64

NC = 2
NS = 16
L = 16

_MESH = dict(core_axis_name="c", subcore_axis_name="s", num_cores=NC,
             num_subcores=NS)

_DEG_W = 4096
_E_PER_TILE_DEG = N_EDGES // (NC * NS)
_N_PER_TILE = N_NODES // NS


def _deg_body(index_hbm, ew_win_hbm, degp_hbm, dacc, zbuf, colbuf, ewb):
  c = lax.axis_index("c")
  s = lax.axis_index("s")

  def zero_vec(i, _):
    zbuf[pl.ds(i * L, L)] = jnp.zeros((L,), jnp.float32)
    return 0
  lax.fori_loop(0, _DEG_W // L, zero_vec, 0)

  def zero_chunk(i, _):
    pltpu.sync_copy(zbuf, dacc.at[pl.ds(s * _N_PER_TILE + i * _DEG_W, _DEG_W)])
    return 0
  lax.fori_loop(0, _N_PER_TILE // _DEG_W, zero_chunk, 0)
  plsc.subcore_barrier()

  pltpu.sync_copy(ew_win_hbm, ewb)
  tile_base = (c * NS + s) * _E_PER_TILE_DEG

  def win(w, _):
    ebase = tile_base + w * _DEG_W
    pltpu.sync_copy(index_hbm.at[1, pl.ds(ebase, _DEG_W)], colbuf)
    pltpu.sync_copy(ewb, dacc.at[colbuf], add=True)
    return 0
  lax.fori_loop(0, _E_PER_TILE_DEG // _DEG_W, win, 0)
  plsc.subcore_barrier()

  pltpu.sync_copy(dacc.at[pl.ds(s * _N_PER_TILE, _N_PER_TILE)],
                  degp_hbm.at[c, pl.ds(s * _N_PER_TILE, _N_PER_TILE)])


def _k_deg(index, ew_win):
  f = pl.kernel(
      _deg_body,
      out_type=jax.ShapeDtypeStruct((NC, N_NODES), jnp.float32),
      mesh=plsc.VectorSubcoreMesh(**_MESH),
      compiler_params=pltpu.CompilerParams(needs_layout_passes=False),
      scratch_types=[
          pltpu.VMEM_SHARED((N_NODES,), jnp.float32),
          pltpu.VMEM((_DEG_W,), jnp.float32),
          pltpu.VMEM((_DEG_W,), jnp.int32),
          pltpu.VMEM((_DEG_W,), jnp.float32),
      ],
  )
  return f(index, ew_win)


_R = 10240
_CHUNKS = 26
_PASSES = _CHUNKS // NC
_HOP_W = 2048
_UNROLL = 1
_E_PER_TILE = N_EDGES // NS
_GK = 256
_LCAP = 2368
_RPT = _R // NS
_TAIL_BASE = (_CHUNKS - 1) * _R
_TAIL_ROWS = N_NODES - _TAIL_BASE
_TAIL_RPT = _TAIL_ROWS // NS
_WPAD = 64
_LCBITS = 14
_LCMASK = (1 << _LCBITS) - 1


def _hop_fire(src_hbm, acc, flr, flc, st_r, st_c, st_w, gbuf, ewb,
              gsem, start, mall):
  def stage(i, _):
    st_r[pl.ds(i * L, L)] = flr[pl.ds(start + i * L, L)]
    pk = flc[pl.ds(start + i * L, L)]
    st_c[pl.ds(i * L, L)] = pk & _LCMASK
    st_w[pl.ds(i * L, L)] = plsc.load_gather(
        ewb, [lax.shift_right_logical(pk, _LCBITS)], mask=mall)
    return 0
  lax.fori_loop(0, _GK // L, stage, 0)

  pltpu.async_copy(src_hbm.at[st_r], gbuf, gsem).wait()

  def scale(g, _):
    wv = st_w[pl.ds(g * L, L)]
    for k in range(L):
      wsp = jnp.full((L,), wv[k], jnp.float32)
      r = g * L + k
      for q in range(NUM_IN // L):
        gbuf[r, pl.ds(q * L, L)] = gbuf[r, pl.ds(q * L, L)] * wsp
    return 0
  lax.fori_loop(0, _GK // L, scale, 0)

  pltpu.sync_copy(gbuf, acc.at[st_c], add=True)


def _hop_body(src_hbm, index_hbm, w64_hbm, dst_hbm,
              acc, flr, flc, st_r, st_c, st_w, gbuf, colw, roww, ewb,
              gsem, wsem):
  c = lax.axis_index("c")
  s = lax.axis_index("s")
  pltpu.sync_copy(w64_hbm, ewb)
  lane = lax.iota(jnp.int32, L)
  mall = lane >= 0
  tile_e = s * _E_PER_TILE
  n_win = _E_PER_TILE // _HOP_W

  def do_pass(p, _):
    k = NC * p + c
    is_tail = k == (_CHUNKS - 1)
    base = jnp.where(is_tail, _TAIL_BASE, k * _R)
    init_base = jnp.where(is_tail, N_NODES - _R, k * _R)
    hi = jnp.where(is_tail, N_NODES, k * _R + _R)

    pltpu.sync_copy(src_hbm.at[pl.ds(init_base + s * _RPT, _RPT)],
                    acc.at[pl.ds(s * _RPT, _RPT)])
    plsc.subcore_barrier()

    def drain(cnt, fired, fctr):
      def one(d, carry):
        fired, fctr = carry
        go = fired + _GK <= cnt
        @pl.when(go)
        def _():
          _hop_fire(src_hbm, acc, flr, flc, st_r, st_c, st_w, gbuf,
                    ewb, gsem, fired, mall)
        adv = jnp.where(go, 1, 0)
        return (fired + adv * _GK, fctr + adv)
      fired, fctr = lax.fori_loop(0, (_LCAP + _GK - 1) // _GK, one,
                                  (fired, fctr))
      rem = cnt - fired
      def mv(i, _):
        @pl.when(i * L < rem)
        def _():
          flr[pl.ds(i * L, L)] = flr[pl.ds(fired + i * L, L)]
          flc[pl.ds(i * L, L)] = flc[pl.ds(fired + i * L, L)]
        return 0
      lax.fori_loop(0, _GK // L, mv, 0)
      return rem, fctr

    def win(w, carry):
      cnt, fctr = carry
      ebase = tile_e + w * _HOP_W
      pltpu.async_copy(index_hbm.at[1, pl.ds(ebase, _HOP_W)], colw, wsem)
      pltpu.async_copy(index_hbm.at[0, pl.ds(ebase, _HOP_W)], roww, wsem)
      pltpu.make_async_copy(index_hbm.at[1, pl.ds(ebase, _HOP_W)], colw,
                            wsem).wait()
      pltpu.make_async_copy(index_hbm.at[0, pl.ds(ebase, _HOP_W)], roww,
                            wsem).wait()

      def scan(jj, cnt):
        for u in range(_UNROLL):
          j = jj * _UNROLL + u
          c16 = colw[pl.ds(j * L, L)]
          m = (c16 >= base) & (c16 < hi)
          r16 = roww[pl.ds(j * L, L)]
          widx16 = lax.rem(j, 4) * L + lane
          pk16 = (c16 - init_base) | lax.shift_left(widx16, _LCBITS)
          cs = plsc.cumsum(m.astype(jnp.int32))
          pos = cs + (cnt - 1)
          plsc.store_scatter(flr, [pos], r16, mask=m)
          plsc.store_scatter(flc, [pos], pk16, mask=m)
          cnt = cnt + cs[L - 1]
        return cnt
      cnt = lax.fori_loop(0, _HOP_W // L // _UNROLL, scan, cnt)
      return drain(cnt, 0, fctr)

    cnt, fctr = lax.fori_loop(0, n_win, win, (0, 0))

    cnt_pad = jnp.where(cnt > 0, ((cnt + _GK - 1) // _GK) * _GK, 0)
    pad_pk = lane | (_WPAD << _LCBITS)
    def pad(i, _):
      @pl.when(cnt + i * L < cnt_pad)
      def _():
        off = cnt + i * L
        flr[pl.ds(off, L)] = lane
        flc[pl.ds(off, L)] = pad_pk
      return 0
    lax.fori_loop(0, _GK // L, pad, 0)
    _, fctr = drain(cnt_pad, 0, fctr)

    plsc.subcore_barrier()
    @pl.when(jnp.logical_not(is_tail))
    def _():
      pltpu.sync_copy(acc.at[pl.ds(s * _RPT, _RPT)],
                      dst_hbm.at[pl.ds(init_base + s * _RPT, _RPT)])
    @pl.when(is_tail)
    def _():
      pltpu.sync_copy(
          acc.at[pl.ds(_R - _TAIL_ROWS + s * _TAIL_RPT, _TAIL_RPT)],
          dst_hbm.at[pl.ds(_TAIL_BASE + s * _TAIL_RPT, _TAIL_RPT)])
    plsc.subcore_barrier()
    return 0

  lax.fori_loop(0, _PASSES, do_pass, 0)


def _k_hop(src, index, w64pad):
  f = pl.kernel(
      _hop_body,
      out_type=jax.ShapeDtypeStruct((N_NODES, NUM_IN), jnp.float32),
      mesh=plsc.VectorSubcoreMesh(**_MESH),
      compiler_params=pltpu.CompilerParams(needs_layout_passes=False),
      scratch_types=[
          pltpu.VMEM_SHARED((_R, NUM_IN), jnp.float32),
          pltpu.VMEM((_LCAP,), jnp.int32),
          pltpu.VMEM((_LCAP,), jnp.int32),
          pltpu.VMEM((_GK,), jnp.int32),
          pltpu.VMEM((_GK,), jnp.int32),
          pltpu.VMEM((_GK,), jnp.float32),
          pltpu.VMEM((_GK, NUM_IN), jnp.float32),
          pltpu.VMEM((_HOP_W,), jnp.int32),
          pltpu.VMEM((_HOP_W,), jnp.int32),
          pltpu.VMEM((_WPAD + 16,), jnp.float32),
          pltpu.SemaphoreType.DMA,
          pltpu.SemaphoreType.DMA,
      ],
  )
  return f(src, index, w64pad)


_POOL_W = 2048
_ROWS_PER_TILE = N_NODES // (NC * NS)
_G_PER_TILE = N_GRAPHS // NS
LG = 16


_PACC = N_GRAPHS * LG
_ZP = _PACC // NS


def _pool_body(lg_hbm, batch_hbm, out_hbm, pacc, zbuf, rbuf, bbuf, ibuf):
  c = lax.axis_index("c")
  s = lax.axis_index("s")
  lane = lax.iota(jnp.int32, L)

  def zero_vec(i, _):
    zbuf[pl.ds(i * L, L)] = jnp.zeros((L,), jnp.float32)
    return 0
  lax.fori_loop(0, _ZP // L, zero_vec, 0)
  pltpu.sync_copy(zbuf, pacc.at[pl.ds(s * _ZP, _ZP)])
  plsc.subcore_barrier()

  tile_base = (c * NS + s) * _ROWS_PER_TILE

  def win(w, _):
    rbase = tile_base + w * _POOL_W
    pltpu.sync_copy(lg_hbm.at[pl.ds(rbase * LG, _POOL_W * LG)], rbuf)
    pltpu.sync_copy(batch_hbm.at[pl.ds(rbase, _POOL_W)], bbuf)

    mall = lane >= 0
    def expand(g, _):
      bv = bbuf[pl.ds(g * L, L)] * LG
      ppos = g * (L * LG) + lane * LG
      for j in range(LG):
        plsc.store_scatter(ibuf, [ppos + j], bv + j, mask=mall)
      return 0
    lax.fori_loop(0, _POOL_W // L, expand, 0)
    pltpu.sync_copy(rbuf, pacc.at[ibuf], add=True)
    return 0
  lax.fori_loop(0, _ROWS_PER_TILE // _POOL_W, win, 0)
  plsc.subcore_barrier()

  pltpu.sync_copy(pacc.at[pl.ds(s * _ZP, _ZP)],
                  out_hbm.at[c, pl.ds(s * _ZP, _ZP)])


def _k_pool(lg_flat, batch):
  f = pl.kernel(
      _pool_body,
      out_type=jax.ShapeDtypeStruct((NC, _PACC), jnp.float32),
      mesh=plsc.VectorSubcoreMesh(**_MESH),
      compiler_params=pltpu.CompilerParams(needs_layout_passes=False),
      scratch_types=[
          pltpu.VMEM_SHARED((_PACC,), jnp.float32),
          pltpu.VMEM((_ZP,), jnp.float32),
          pltpu.VMEM((_POOL_W * LG,), jnp.float32),
          pltpu.VMEM((_POOL_W,), jnp.int32),
          pltpu.VMEM((_POOL_W * LG,), jnp.int32),
      ],
  )
  return f(lg_flat, batch)


_PRE_B = 2048
_NR = N_NODES // NUM_IN
_PRE_R = _PRE_B // NUM_IN


def _pre_body(degp_ref, x_ref, dinv_ref, dinv2_ref, y0_ref):
  deg = 1.0 + degp_ref[0] + degp_ref[1]
  dinv = lax.rsqrt(deg)
  dinv_ref[...] = dinv
  dinv2_ref[...] = 1.0 / deg
  y0_ref[...] = x_ref[...] * dinv[:, :, None]


def _k_pre(degp, x):
  grid = N_NODES // _PRE_B
  return pl.pallas_call(
      _pre_body,
      grid=(grid,),
      in_specs=[
          pl.BlockSpec((NC, _PRE_R, NUM_IN), lambda i: (0, i, 0)),
          pl.BlockSpec((_PRE_R, NUM_IN, NUM_IN), lambda i: (i, 0, 0)),
      ],
      out_specs=[
          pl.BlockSpec((_PRE_R, NUM_IN), lambda i: (i, 0)),
          pl.BlockSpec((_PRE_R, NUM_IN), lambda i: (i, 0)),
          pl.BlockSpec((_PRE_R, NUM_IN, NUM_IN), lambda i: (i, 0, 0)),
      ],
      out_shape=[
          jax.ShapeDtypeStruct((_NR, NUM_IN), jnp.float32),
          jax.ShapeDtypeStruct((_NR, NUM_IN), jnp.float32),
          jax.ShapeDtypeStruct((_NR, NUM_IN, NUM_IN), jnp.float32),
      ],
  )(degp.reshape(NC, _NR, NUM_IN), x.reshape(_NR, NUM_IN, NUM_IN))


def _scale_body(y_ref, d_ref, o_ref):
  o_ref[...] = y_ref[...] * d_ref[...][:, :, None]


def _k_scale(y3d, d2d):
  grid = _NR // _PRE_R
  return pl.pallas_call(
      _scale_body,
      grid=(grid,),
      in_specs=[
          pl.BlockSpec((_PRE_R, NUM_IN, NUM_IN), lambda i: (i, 0, 0)),
          pl.BlockSpec((_PRE_R, NUM_IN), lambda i: (i, 0)),
      ],
      out_specs=pl.BlockSpec((_PRE_R, NUM_IN, NUM_IN), lambda i: (i, 0, 0)),
      out_shape=jax.ShapeDtypeStruct((_NR, NUM_IN, NUM_IN), jnp.float32),
  )(y3d, d2d)


_MM_B = 2048
_MM_R = _MM_B // NUM_IN


def _mm_body(y3_ref, dinv_ref, linwT_ref, linb_ref, fcwT_ref, o_ref):
  h2 = (y3_ref[...] * dinv_ref[...][:, :, None]).reshape(_MM_B, NUM_IN)
  h = jnp.dot(h2, linwT_ref[...], preferred_element_type=jnp.float32)
  h = jnp.maximum(h + linb_ref[...], 0.0)
  o_ref[...] = jnp.dot(h, fcwT_ref[...], preferred_element_type=jnp.float32)


def _k_mm(y3_3d, dinv2d, linwT, linb, fcwT):
  grid = N_NODES // _MM_B
  return pl.pallas_call(
      _mm_body,
      grid=(grid,),
      in_specs=[
          pl.BlockSpec((_MM_R, NUM_IN, NUM_IN), lambda i: (i, 0, 0)),
          pl.BlockSpec((_MM_R, NUM_IN), lambda i: (i, 0)),
          pl.BlockSpec((NUM_IN, NUM_HIDDEN), lambda i: (0, 0)),
          pl.BlockSpec((1, NUM_HIDDEN), lambda i: (0, 0)),
          pl.BlockSpec((NUM_HIDDEN, LG), lambda i: (0, 0)),
      ],
      out_specs=pl.BlockSpec((_MM_B, LG), lambda i: (i, 0)),
      out_shape=jax.ShapeDtypeStruct((N_NODES, LG), jnp.float32),
  )(y3_3d, dinv2d, linwT, linb, fcwT)


def _soft_body(pp_ref, fcb_ref, o_ref):
  z = pp_ref[0] + pp_ref[1] + fcb_ref[...]
  col = lax.broadcasted_iota(jnp.int32, (N_GRAPHS, LG), 1)
  valid = col < NUM_CLASS
  z = jnp.where(valid, z, -1e30)
  z = z - jnp.max(z, axis=1, keepdims=True)
  p = jnp.exp(z)
  p = jnp.where(valid, p, 0.0)
  o_ref[...] = p / jnp.sum(p, axis=1, keepdims=True)


def _k_soft(pooledp, fcb):
  return pl.pallas_call(
      _soft_body,
      in_specs=[
          pl.BlockSpec((NC, N_GRAPHS, LG), lambda: (0, 0, 0)),
          pl.BlockSpec((1, LG), lambda: (0, 0)),
      ],
      out_specs=pl.BlockSpec((N_GRAPHS, LG), lambda: (0, 0)),
      out_shape=jax.ShapeDtypeStruct((N_GRAPHS, LG), jnp.float32),
  )(pooledp, fcb)


def kernel(x, index, batch, weight, lin_w, lin_b, fc_w, fc_b):
  index = index.astype(jnp.int32)
  batch = batch.astype(jnp.int32)
  ew_win = jnp.tile(weight, _DEG_W // E_PER_GRAPH)
  w64pad = jnp.zeros((_WPAD + 16,), jnp.float32).at[:E_PER_GRAPH].set(weight)

  degp = _k_deg(index, ew_win)
  dinv, dinv2, y0_3d = _k_pre(degp, x)
  y1 = _k_hop(y0_3d.reshape(N_NODES, NUM_IN), index, w64pad)
  y2_3d = _k_scale(y1.reshape(_NR, NUM_IN, NUM_IN), dinv2)
  y3 = _k_hop(y2_3d.reshape(N_NODES, NUM_IN), index, w64pad)

  linwT = lin_w.T
  linb = lin_b.reshape(1, NUM_HIDDEN)
  fcwT = jnp.zeros((NUM_HIDDEN, LG), jnp.float32).at[:, :NUM_CLASS].set(fc_w.T)
  lg = _k_mm(y3.reshape(_NR, NUM_IN, NUM_IN), dinv, linwT, linb, fcwT)

  pooledp = _k_pool(lg.reshape(-1), batch)
  fcb = jnp.zeros((1, LG), jnp.float32).at[0, :NUM_CLASS].set(fc_b)
  probs = _k_soft(pooledp.reshape(NC, N_GRAPHS, LG), fcb)
  return probs[:, :NUM_CLASS]


# speedup vs baseline: 1.5137x; 1.0001x over previous
import functools

import jax
import jax.numpy as jnp
from jax import lax
from jax.experimental import pallas as pl
from jax.experimental.pallas import tpu as pltpu
import jax.experimental.pallas.tpu_sc as plsc

N_NODES = 262144
N_EDGES = 524288
NUM_IN = 128
NUM_HIDDEN = 256
NUM_CLASS = 10
N_GRAPHS = 8192
E_PER_GRAPH = 64

NC = 2
NS = 16
L = 16

_MESH = dict(core_axis_name="c", subcore_axis_name="s", num_cores=NC,
             num_subcores=NS)

_DEG_W = 4096
_E_PER_TILE_DEG = N_EDGES // (NC * NS)
_N_PER_TILE = N_NODES // NS


def _deg_body(index_hbm, ew_win_hbm, degp_hbm, dacc, zbuf, colbuf, ewb):
  c = lax.axis_index("c")
  s = lax.axis_index("s")

  def zero_vec(i, _):
    zbuf[pl.ds(i * L, L)] = jnp.zeros((L,), jnp.float32)
    return 0
  lax.fori_loop(0, _DEG_W // L, zero_vec, 0)

  def zero_chunk(i, _):
    pltpu.sync_copy(zbuf, dacc.at[pl.ds(s * _N_PER_TILE + i * _DEG_W, _DEG_W)])
    return 0
  lax.fori_loop(0, _N_PER_TILE // _DEG_W, zero_chunk, 0)
  plsc.subcore_barrier()

  pltpu.sync_copy(ew_win_hbm, ewb)
  tile_base = (c * NS + s) * _E_PER_TILE_DEG

  def win(w, _):
    ebase = tile_base + w * _DEG_W
    pltpu.sync_copy(index_hbm.at[1, pl.ds(ebase, _DEG_W)], colbuf)
    pltpu.sync_copy(ewb, dacc.at[colbuf], add=True)
    return 0
  lax.fori_loop(0, _E_PER_TILE_DEG // _DEG_W, win, 0)
  plsc.subcore_barrier()

  pltpu.sync_copy(dacc.at[pl.ds(s * _N_PER_TILE, _N_PER_TILE)],
                  degp_hbm.at[c, pl.ds(s * _N_PER_TILE, _N_PER_TILE)])


def _k_deg(index, ew_win):
  f = pl.kernel(
      _deg_body,
      out_type=jax.ShapeDtypeStruct((NC, N_NODES), jnp.float32),
      mesh=plsc.VectorSubcoreMesh(**_MESH),
      compiler_params=pltpu.CompilerParams(needs_layout_passes=False),
      scratch_types=[
          pltpu.VMEM_SHARED((N_NODES,), jnp.float32),
          pltpu.VMEM((_DEG_W,), jnp.float32),
          pltpu.VMEM((_DEG_W,), jnp.int32),
          pltpu.VMEM((_DEG_W,), jnp.float32),
      ],
  )
  return f(index, ew_win)


_R = 10240
_CHUNKS = 26
_PASSES = _CHUNKS // NC
_HOP_W = 2048
_UNROLL = 2
_E_PER_TILE = N_EDGES // NS
_GK = 256
_LCAP = 2368
_RPT = _R // NS
_TAIL_BASE = (_CHUNKS - 1) * _R
_TAIL_ROWS = N_NODES - _TAIL_BASE
_TAIL_RPT = _TAIL_ROWS // NS
_WPAD = 64
_LCBITS = 14
_LCMASK = (1 << _LCBITS) - 1


def _hop_fire(src_hbm, acc, flr, flc, st_r, st_c, st_w, gbuf, ewb,
              gsem, start, mall):
  def stage(i, _):
    st_r[pl.ds(i * L, L)] = flr[pl.ds(start + i * L, L)]
    pk = flc[pl.ds(start + i * L, L)]
    st_c[pl.ds(i * L, L)] = pk & _LCMASK
    st_w[pl.ds(i * L, L)] = plsc.load_gather(
        ewb, [lax.shift_right_logical(pk, _LCBITS)], mask=mall)
    return 0
  lax.fori_loop(0, _GK // L, stage, 0)

  pltpu.async_copy(src_hbm.at[st_r], gbuf, gsem).wait()

  def scale(g, _):
    wv = st_w[pl.ds(g * L, L)]
    for k in range(L):
      wsp = jnp.full((L,), wv[k], jnp.float32)
      r = g * L + k
      for q in range(NUM_IN // L):
        gbuf[r, pl.ds(q * L, L)] = gbuf[r, pl.ds(q * L, L)] * wsp
    return 0
  lax.fori_loop(0, _GK // L, scale, 0)

  pltpu.sync_copy(gbuf, acc.at[st_c], add=True)


def _hop_body(src_hbm, index_hbm, w64_hbm, dst_hbm,
              acc, flr, flc, st_r, st_c, st_w, gbuf, colw, roww, ewb,
              gsem, wsem):
  c = lax.axis_index("c")
  s = lax.axis_index("s")
  pltpu.sync_copy(w64_hbm, ewb)
  lane = lax.iota(jnp.int32, L)
  mall = lane >= 0
  tile_e = s * _E_PER_TILE
  n_win = _E_PER_TILE // _HOP_W

  def do_pass(p, _):
    k = NC * p + c
    is_tail = k == (_CHUNKS - 1)
    base = jnp.where(is_tail, _TAIL_BASE, k * _R)
    init_base = jnp.where(is_tail, N_NODES - _R, k * _R)
    hi = jnp.where(is_tail, N_NODES, k * _R + _R)

    pltpu.sync_copy(src_hbm.at[pl.ds(init_base + s * _RPT, _RPT)],
                    acc.at[pl.ds(s * _RPT, _RPT)])
    plsc.subcore_barrier()

    def drain(cnt, fired, fctr):
      def one(d, carry):
        fired, fctr = carry
        go = fired + _GK <= cnt
        @pl.when(go)
        def _():
          _hop_fire(src_hbm, acc, flr, flc, st_r, st_c, st_w, gbuf,
                    ewb, gsem, fired, mall)
        adv = jnp.where(go, 1, 0)
        return (fired + adv * _GK, fctr + adv)
      fired, fctr = lax.fori_loop(0, (_LCAP + _GK - 1) // _GK, one,
                                  (fired, fctr))
      rem = cnt - fired
      def mv(i, _):
        @pl.when(i * L < rem)
        def _():
          flr[pl.ds(i * L, L)] = flr[pl.ds(fired + i * L, L)]
          flc[pl.ds(i * L, L)] = flc[pl.ds(fired + i * L, L)]
        return 0
      lax.fori_loop(0, _GK // L, mv, 0)
      return rem, fctr

    def win(w, carry):
      cnt, fctr = carry
      ebase = tile_e + w * _HOP_W
      pltpu.async_copy(index_hbm.at[1, pl.ds(ebase, _HOP_W)], colw, wsem)
      pltpu.async_copy(index_hbm.at[0, pl.ds(ebase, _HOP_W)], roww, wsem)
      pltpu.make_async_copy(index_hbm.at[1, pl.ds(ebase, _HOP_W)], colw,
                            wsem).wait()
      pltpu.make_async_copy(index_hbm.at[0, pl.ds(ebase, _HOP_W)], roww,
                            wsem).wait()

      def scan(jj, cnt):
        for u in range(_UNROLL):
          j = jj * _UNROLL + u
          c16 = colw[pl.ds(j * L, L)]
          m = (c16 >= base) & (c16 < hi)
          r16 = roww[pl.ds(j * L, L)]
          widx16 = lax.rem(j, 4) * L + lane
          pk16 = (c16 - init_base) | lax.shift_left(widx16, _LCBITS)
          cs = plsc.cumsum(m.astype(jnp.int32))
          pos = cs + (cnt - 1)
          plsc.store_scatter(flr, [pos], r16, mask=m)
          plsc.store_scatter(flc, [pos], pk16, mask=m)
          cnt = cnt + cs[L - 1]
        return cnt
      cnt = lax.fori_loop(0, _HOP_W // L // _UNROLL, scan, cnt)
      return drain(cnt, 0, fctr)

    cnt, fctr = lax.fori_loop(0, n_win, win, (0, 0))

    cnt_pad = jnp.where(cnt > 0, ((cnt + _GK - 1) // _GK) * _GK, 0)
    pad_pk = lane | (_WPAD << _LCBITS)
    def pad(i, _):
      @pl.when(cnt + i * L < cnt_pad)
      def _():
        off = cnt + i * L
        flr[pl.ds(off, L)] = lane
        flc[pl.ds(off, L)] = pad_pk
      return 0
    lax.fori_loop(0, _GK // L, pad, 0)
    _, fctr = drain(cnt_pad, 0, fctr)

    plsc.subcore_barrier()
    @pl.when(jnp.logical_not(is_tail))
    def _():
      pltpu.sync_copy(acc.at[pl.ds(s * _RPT, _RPT)],
                      dst_hbm.at[pl.ds(init_base + s * _RPT, _RPT)])
    @pl.when(is_tail)
    def _():
      pltpu.sync_copy(
          acc.at[pl.ds(_R - _TAIL_ROWS + s * _TAIL_RPT, _TAIL_RPT)],
          dst_hbm.at[pl.ds(_TAIL_BASE + s * _TAIL_RPT, _TAIL_RPT)])
    plsc.subcore_barrier()
    return 0

  lax.fori_loop(0, _PASSES, do_pass, 0)


def _k_hop(src, index, w64pad):
  f = pl.kernel(
      _hop_body,
      out_type=jax.ShapeDtypeStruct((N_NODES, NUM_IN), jnp.float32),
      mesh=plsc.VectorSubcoreMesh(**_MESH),
      compiler_params=pltpu.CompilerParams(needs_layout_passes=False),
      scratch_types=[
          pltpu.VMEM_SHARED((_R, NUM_IN), jnp.float32),
          pltpu.VMEM((_LCAP,), jnp.int32),
          pltpu.VMEM((_LCAP,), jnp.int32),
          pltpu.VMEM((_GK,), jnp.int32),
          pltpu.VMEM((_GK,), jnp.int32),
          pltpu.VMEM((_GK,), jnp.float32),
          pltpu.VMEM((_GK, NUM_IN), jnp.float32),
          pltpu.VMEM((_HOP_W,), jnp.int32),
          pltpu.VMEM((_HOP_W,), jnp.int32),
          pltpu.VMEM((_WPAD + 16,), jnp.float32),
          pltpu.SemaphoreType.DMA,
          pltpu.SemaphoreType.DMA,
      ],
  )
  return f(src, index, w64pad)


_POOL_W = 2048
_ROWS_PER_TILE = N_NODES // (NC * NS)
_G_PER_TILE = N_GRAPHS // NS
LG = 16


_PACC = N_GRAPHS * LG
_ZP = _PACC // NS


def _pool_body(lg_hbm, batch_hbm, out_hbm, pacc, zbuf, rbuf, bbuf, ibuf):
  c = lax.axis_index("c")
  s = lax.axis_index("s")
  lane = lax.iota(jnp.int32, L)

  def zero_vec(i, _):
    zbuf[pl.ds(i * L, L)] = jnp.zeros((L,), jnp.float32)
    return 0
  lax.fori_loop(0, _ZP // L, zero_vec, 0)
  pltpu.sync_copy(zbuf, pacc.at[pl.ds(s * _ZP, _ZP)])
  plsc.subcore_barrier()

  tile_base = (c * NS + s) * _ROWS_PER_TILE

  def win(w, _):
    rbase = tile_base + w * _POOL_W
    pltpu.sync_copy(lg_hbm.at[pl.ds(rbase * LG, _POOL_W * LG)], rbuf)
    pltpu.sync_copy(batch_hbm.at[pl.ds(rbase, _POOL_W)], bbuf)

    mall = lane >= 0
    def expand(g, _):
      bv = bbuf[pl.ds(g * L, L)] * LG
      ppos = g * (L * LG) + lane * LG
      for j in range(LG):
        plsc.store_scatter(ibuf, [ppos + j], bv + j, mask=mall)
      return 0
    lax.fori_loop(0, _POOL_W // L, expand, 0)
    pltpu.sync_copy(rbuf, pacc.at[ibuf], add=True)
    return 0
  lax.fori_loop(0, _ROWS_PER_TILE // _POOL_W, win, 0)
  plsc.subcore_barrier()

  pltpu.sync_copy(pacc.at[pl.ds(s * _ZP, _ZP)],
                  out_hbm.at[c, pl.ds(s * _ZP, _ZP)])


def _k_pool(lg_flat, batch):
  f = pl.kernel(
      _pool_body,
      out_type=jax.ShapeDtypeStruct((NC, _PACC), jnp.float32),
      mesh=plsc.VectorSubcoreMesh(**_MESH),
      compiler_params=pltpu.CompilerParams(needs_layout_passes=False),
      scratch_types=[
          pltpu.VMEM_SHARED((_PACC,), jnp.float32),
          pltpu.VMEM((_ZP,), jnp.float32),
          pltpu.VMEM((_POOL_W * LG,), jnp.float32),
          pltpu.VMEM((_POOL_W,), jnp.int32),
          pltpu.VMEM((_POOL_W * LG,), jnp.int32),
      ],
  )
  return f(lg_flat, batch)


_PRE_B = 2048
_NR = N_NODES // NUM_IN
_PRE_R = _PRE_B // NUM_IN


def _pre_body(degp_ref, x_ref, dinv_ref, dinv2_ref, y0_ref):
  deg = 1.0 + degp_ref[0] + degp_ref[1]
  dinv = lax.rsqrt(deg)
  dinv_ref[...] = dinv
  dinv2_ref[...] = 1.0 / deg
  y0_ref[...] = x_ref[...] * dinv[:, :, None]


def _k_pre(degp, x):
  grid = N_NODES // _PRE_B
  return pl.pallas_call(
      _pre_body,
      grid=(grid,),
      in_specs=[
          pl.BlockSpec((NC, _PRE_R, NUM_IN), lambda i: (0, i, 0)),
          pl.BlockSpec((_PRE_R, NUM_IN, NUM_IN), lambda i: (i, 0, 0)),
      ],
      out_specs=[
          pl.BlockSpec((_PRE_R, NUM_IN), lambda i: (i, 0)),
          pl.BlockSpec((_PRE_R, NUM_IN), lambda i: (i, 0)),
          pl.BlockSpec((_PRE_R, NUM_IN, NUM_IN), lambda i: (i, 0, 0)),
      ],
      out_shape=[
          jax.ShapeDtypeStruct((_NR, NUM_IN), jnp.float32),
          jax.ShapeDtypeStruct((_NR, NUM_IN), jnp.float32),
          jax.ShapeDtypeStruct((_NR, NUM_IN, NUM_IN), jnp.float32),
      ],
  )(degp.reshape(NC, _NR, NUM_IN), x.reshape(_NR, NUM_IN, NUM_IN))


def _scale_body(y_ref, d_ref, o_ref):
  o_ref[...] = y_ref[...] * d_ref[...][:, :, None]


def _k_scale(y3d, d2d):
  grid = _NR // _PRE_R
  return pl.pallas_call(
      _scale_body,
      grid=(grid,),
      in_specs=[
          pl.BlockSpec((_PRE_R, NUM_IN, NUM_IN), lambda i: (i, 0, 0)),
          pl.BlockSpec((_PRE_R, NUM_IN), lambda i: (i, 0)),
      ],
      out_specs=pl.BlockSpec((_PRE_R, NUM_IN, NUM_IN), lambda i: (i, 0, 0)),
      out_shape=jax.ShapeDtypeStruct((_NR, NUM_IN, NUM_IN), jnp.float32),
  )(y3d, d2d)


_MM_B = 2048
_MM_R = _MM_B // NUM_IN


def _mm_body(y3_ref, dinv_ref, linwT_ref, linb_ref, fcwT_ref, o_ref):
  h2 = (y3_ref[...] * dinv_ref[...][:, :, None]).reshape(_MM_B, NUM_IN)
  h = jnp.dot(h2, linwT_ref[...], preferred_element_type=jnp.float32)
  h = jnp.maximum(h + linb_ref[...], 0.0)
  o_ref[...] = jnp.dot(h, fcwT_ref[...], preferred_element_type=jnp.float32)


def _k_mm(y3_3d, dinv2d, linwT, linb, fcwT):
  grid = N_NODES // _MM_B
  return pl.pallas_call(
      _mm_body,
      grid=(grid,),
      in_specs=[
          pl.BlockSpec((_MM_R, NUM_IN, NUM_IN), lambda i: (i, 0, 0)),
          pl.BlockSpec((_MM_R, NUM_IN), lambda i: (i, 0)),
          pl.BlockSpec((NUM_IN, NUM_HIDDEN), lambda i: (0, 0)),
          pl.BlockSpec((1, NUM_HIDDEN), lambda i: (0, 0)),
          pl.BlockSpec((NUM_HIDDEN, LG), lambda i: (0, 0)),
      ],
      out_specs=pl.BlockSpec((_MM_B, LG), lambda i: (i, 0)),
      out_shape=jax.ShapeDtypeStruct((N_NODES, LG), jnp.float32),
  )(y3_3d, dinv2d, linwT, linb, fcwT)


def _soft_body(pp_ref, fcb_ref, o_ref):
  z = pp_ref[0] + pp_ref[1] + fcb_ref[...]
  col = lax.broadcasted_iota(jnp.int32, (N_GRAPHS, LG), 1)
  valid = col < NUM_CLASS
  z = jnp.where(valid, z, -1e30)
  z = z - jnp.max(z, axis=1, keepdims=True)
  p = jnp.exp(z)
  p = jnp.where(valid, p, 0.0)
  o_ref[...] = p / jnp.sum(p, axis=1, keepdims=True)


def _k_soft(pooledp, fcb):
  return pl.pallas_call(
      _soft_body,
      in_specs=[
          pl.BlockSpec((NC, N_GRAPHS, LG), lambda: (0, 0, 0)),
          pl.BlockSpec((1, LG), lambda: (0, 0)),
      ],
      out_specs=pl.BlockSpec((N_GRAPHS, LG), lambda: (0, 0)),
      out_shape=jax.ShapeDtypeStruct((N_GRAPHS, LG), jnp.float32),
  )(pooledp, fcb)


def kernel(x, index, batch, weight, lin_w, lin_b, fc_w, fc_b):
  index = index.astype(jnp.int32)
  batch = batch.astype(jnp.int32)
  ew_win = jnp.tile(weight, _DEG_W // E_PER_GRAPH)
  w64pad = jnp.zeros((_WPAD + 16,), jnp.float32).at[:E_PER_GRAPH].set(weight)

  degp = _k_deg(index, ew_win)
  dinv, dinv2, y0_3d = _k_pre(degp, x)
  y1 = _k_hop(y0_3d.reshape(N_NODES, NUM_IN), index, w64pad)
  y2_3d = _k_scale(y1.reshape(_NR, NUM_IN, NUM_IN), dinv2)
  y3 = _k_hop(y2_3d.reshape(N_NODES, NUM_IN), index, w64pad)

  linwT = lin_w.T
  linb = lin_b.reshape(1, NUM_HIDDEN)
  fcwT = jnp.zeros((NUM_HIDDEN, LG), jnp.float32).at[:, :NUM_CLASS].set(fc_w.T)
  lg = _k_mm(y3.reshape(_NR, NUM_IN, NUM_IN), dinv, linwT, linb, fcwT)

  pooledp = _k_pool(lg.reshape(-1), batch)
  fcb = jnp.zeros((1, LG), jnp.float32).at[0, :NUM_CLASS].set(fc_b)
  probs = _k_soft(pooledp.reshape(NC, N_GRAPHS, LG), fcb)
  return probs[:, :NUM_CLASS]


# speedup vs baseline: 1.7483x; 1.1550x over previous
import functools

import jax
import jax.numpy as jnp
from jax import lax
from jax.experimental import pallas as pl
from jax.experimental.pallas import tpu as pltpu
import jax.experimental.pallas.tpu_sc as plsc

N_NODES = 262144
N_EDGES = 524288
NUM_IN = 128
NUM_HIDDEN = 256
NUM_CLASS = 10
N_GRAPHS = 8192
E_PER_GRAPH = 64

NC = 2
NS = 16
L = 16

_MESH = dict(core_axis_name="c", subcore_axis_name="s", num_cores=NC,
             num_subcores=NS)

_DEG_W = 4096
_E_PER_TILE_DEG = N_EDGES // (NC * NS)
_N_PER_TILE = N_NODES // NS


def _deg_body(index_hbm, ew_win_hbm, degp_hbm, dacc, zbuf, colbuf, ewb):
  c = lax.axis_index("c")
  s = lax.axis_index("s")

  def zero_vec(i, _):
    zbuf[pl.ds(i * L, L)] = jnp.zeros((L,), jnp.float32)
    return 0
  lax.fori_loop(0, _DEG_W // L, zero_vec, 0)

  def zero_chunk(i, _):
    pltpu.sync_copy(zbuf, dacc.at[pl.ds(s * _N_PER_TILE + i * _DEG_W, _DEG_W)])
    return 0
  lax.fori_loop(0, _N_PER_TILE // _DEG_W, zero_chunk, 0)
  plsc.subcore_barrier()

  pltpu.sync_copy(ew_win_hbm, ewb)
  tile_base = (c * NS + s) * _E_PER_TILE_DEG

  def win(w, _):
    ebase = tile_base + w * _DEG_W
    pltpu.sync_copy(index_hbm.at[1, pl.ds(ebase, _DEG_W)], colbuf)
    pltpu.sync_copy(ewb, dacc.at[colbuf], add=True)
    return 0
  lax.fori_loop(0, _E_PER_TILE_DEG // _DEG_W, win, 0)
  plsc.subcore_barrier()

  pltpu.sync_copy(dacc.at[pl.ds(s * _N_PER_TILE, _N_PER_TILE)],
                  degp_hbm.at[c, pl.ds(s * _N_PER_TILE, _N_PER_TILE)])


def _k_deg(index, ew_win):
  f = pl.kernel(
      _deg_body,
      out_type=jax.ShapeDtypeStruct((NC, N_NODES), jnp.float32),
      mesh=plsc.VectorSubcoreMesh(**_MESH),
      compiler_params=pltpu.CompilerParams(needs_layout_passes=False),
      scratch_types=[
          pltpu.VMEM_SHARED((N_NODES,), jnp.float32),
          pltpu.VMEM((_DEG_W,), jnp.float32),
          pltpu.VMEM((_DEG_W,), jnp.int32),
          pltpu.VMEM((_DEG_W,), jnp.float32),
      ],
  )
  return f(index, ew_win)


_R = 10240
_CHUNKS = 26
_PASSES = _CHUNKS // NC
_HOP_W = 2048
_UNROLL = 1
_E_PER_TILE = N_EDGES // NS
_GK = 128
_LCAP = 2208
_RPT = _R // NS
_TAIL_BASE = (_CHUNKS - 1) * _R
_TAIL_ROWS = N_NODES - _TAIL_BASE
_TAIL_RPT = _TAIL_ROWS // NS
_WPAD = 64
_LCBITS = 14
_LCMASK = (1 << _LCBITS) - 1


def _hop_stage(flr, flc, st_r, st_c, st_w, ewb, start, mall):
  def stage(i, _):
    st_r[pl.ds(i * L, L)] = flr[pl.ds(start + i * L, L)]
    pk = flc[pl.ds(start + i * L, L)]
    st_c[pl.ds(i * L, L)] = pk & _LCMASK
    st_w[pl.ds(i * L, L)] = plsc.load_gather(
        ewb, [lax.shift_right_logical(pk, _LCBITS)], mask=mall)
    return 0
  lax.fori_loop(0, _GK // L, stage, 0)


def _hop_finish(src_hbm, acc, st_r, st_c, st_w, gbuf, gsem):
  pltpu.make_async_copy(src_hbm.at[st_r], gbuf, gsem).wait()

  def scale(g, _):
    wv = st_w[pl.ds(g * L, L)]
    for k in range(L):
      wsp = jnp.full((L,), wv[k], jnp.float32)
      r = g * L + k
      for q in range(NUM_IN // L):
        gbuf[r, pl.ds(q * L, L)] = gbuf[r, pl.ds(q * L, L)] * wsp
    return 0
  lax.fori_loop(0, _GK // L, scale, 0)

  pltpu.sync_copy(gbuf, acc.at[st_c], add=True)


def _hop_fire(src_hbm, acc, flr, flc, stA, stB, gbufA, gbufB, ewb,
              gsemA, gsemB, start, fctr, mall):
  par = lax.rem(fctr, 2)
  @pl.when(par == 0)
  def _():
    _hop_stage(flr, flc, stA[0], stA[1], stA[2], ewb, start, mall)
    pltpu.async_copy(src_hbm.at[stA[0]], gbufA, gsemA)
  @pl.when(par == 1)
  def _():
    _hop_stage(flr, flc, stB[0], stB[1], stB[2], ewb, start, mall)
    pltpu.async_copy(src_hbm.at[stB[0]], gbufB, gsemB)
  @pl.when((fctr >= 1) & (par == 1))
  def _():
    _hop_finish(src_hbm, acc, stA[0], stA[1], stA[2], gbufA, gsemA)
  @pl.when((fctr >= 1) & (par == 0))
  def _():
    _hop_finish(src_hbm, acc, stB[0], stB[1], stB[2], gbufB, gsemB)


def _hop_body(src_hbm, index_hbm, w64_hbm, dst_hbm,
              acc, flr, flc, st_rA, st_cA, st_wA, gbufA,
              st_rB, st_cB, st_wB, gbufB, colw, roww, ewb,
              gsemA, gsemB, wsem):
  stA = (st_rA, st_cA, st_wA)
  stB = (st_rB, st_cB, st_wB)
  c = lax.axis_index("c")
  s = lax.axis_index("s")
  pltpu.sync_copy(w64_hbm, ewb)
  lane = lax.iota(jnp.int32, L)
  mall = lane >= 0
  tile_e = s * _E_PER_TILE
  n_win = _E_PER_TILE // _HOP_W

  def do_pass(p, _):
    k = NC * p + c
    is_tail = k == (_CHUNKS - 1)
    base = jnp.where(is_tail, _TAIL_BASE, k * _R)
    init_base = jnp.where(is_tail, N_NODES - _R, k * _R)
    hi = jnp.where(is_tail, N_NODES, k * _R + _R)

    pltpu.sync_copy(src_hbm.at[pl.ds(init_base + s * _RPT, _RPT)],
                    acc.at[pl.ds(s * _RPT, _RPT)])
    plsc.subcore_barrier()

    def drain(cnt, fired, fctr):
      def one(d, carry):
        fired, fctr = carry
        go = fired + _GK <= cnt
        @pl.when(go)
        def _():
          _hop_fire(src_hbm, acc, flr, flc, stA, stB, gbufA, gbufB,
                    ewb, gsemA, gsemB, fired, fctr, mall)
        adv = jnp.where(go, 1, 0)
        return (fired + adv * _GK, fctr + adv)
      fired, fctr = lax.fori_loop(0, (_LCAP + _GK - 1) // _GK, one,
                                  (fired, fctr))
      rem = cnt - fired
      def mv(i, _):
        @pl.when(i * L < rem)
        def _():
          flr[pl.ds(i * L, L)] = flr[pl.ds(fired + i * L, L)]
          flc[pl.ds(i * L, L)] = flc[pl.ds(fired + i * L, L)]
        return 0
      lax.fori_loop(0, _GK // L, mv, 0)
      return rem, fctr

    def win(w, carry):
      cnt, fctr = carry
      ebase = tile_e + w * _HOP_W
      pltpu.async_copy(index_hbm.at[1, pl.ds(ebase, _HOP_W)], colw, wsem)
      pltpu.async_copy(index_hbm.at[0, pl.ds(ebase, _HOP_W)], roww, wsem)
      pltpu.make_async_copy(index_hbm.at[1, pl.ds(ebase, _HOP_W)], colw,
                            wsem).wait()
      pltpu.make_async_copy(index_hbm.at[0, pl.ds(ebase, _HOP_W)], roww,
                            wsem).wait()

      def scan(jj, cnt):
        for u in range(_UNROLL):
          j = jj * _UNROLL + u
          c16 = colw[pl.ds(j * L, L)]
          m = (c16 >= base) & (c16 < hi)
          r16 = roww[pl.ds(j * L, L)]
          widx16 = lax.rem(j, 4) * L + lane
          pk16 = (c16 - init_base) | lax.shift_left(widx16, _LCBITS)
          cs = plsc.cumsum(m.astype(jnp.int32))
          pos = cs + (cnt - 1)
          plsc.store_scatter(flr, [pos], r16, mask=m)
          plsc.store_scatter(flc, [pos], pk16, mask=m)
          cnt = cnt + cs[L - 1]
        return cnt
      cnt = lax.fori_loop(0, _HOP_W // L // _UNROLL, scan, cnt)
      return drain(cnt, 0, fctr)

    cnt, fctr = lax.fori_loop(0, n_win, win, (0, 0))

    cnt_pad = jnp.where(cnt > 0, ((cnt + _GK - 1) // _GK) * _GK, 0)
    pad_pk = lane | (_WPAD << _LCBITS)
    def pad(i, _):
      @pl.when(cnt + i * L < cnt_pad)
      def _():
        off = cnt + i * L
        flr[pl.ds(off, L)] = lane
        flc[pl.ds(off, L)] = pad_pk
      return 0
    lax.fori_loop(0, _GK // L, pad, 0)
    _, fctr = drain(cnt_pad, 0, fctr)

    last = lax.rem(fctr - 1, 2)
    @pl.when((fctr >= 1) & (last == 0))
    def _():
      _hop_finish(src_hbm, acc, stA[0], stA[1], stA[2], gbufA, gsemA)
    @pl.when((fctr >= 1) & (last == 1))
    def _():
      _hop_finish(src_hbm, acc, stB[0], stB[1], stB[2], gbufB, gsemB)

    plsc.subcore_barrier()
    @pl.when(jnp.logical_not(is_tail))
    def _():
      pltpu.sync_copy(acc.at[pl.ds(s * _RPT, _RPT)],
                      dst_hbm.at[pl.ds(init_base + s * _RPT, _RPT)])
    @pl.when(is_tail)
    def _():
      pltpu.sync_copy(
          acc.at[pl.ds(_R - _TAIL_ROWS + s * _TAIL_RPT, _TAIL_RPT)],
          dst_hbm.at[pl.ds(_TAIL_BASE + s * _TAIL_RPT, _TAIL_RPT)])
    plsc.subcore_barrier()
    return 0

  lax.fori_loop(0, _PASSES, do_pass, 0)


def _k_hop(src, index, w64pad):
  f = pl.kernel(
      _hop_body,
      out_type=jax.ShapeDtypeStruct((N_NODES, NUM_IN), jnp.float32),
      mesh=plsc.VectorSubcoreMesh(**_MESH),
      compiler_params=pltpu.CompilerParams(needs_layout_passes=False),
      scratch_types=[
          pltpu.VMEM_SHARED((_R, NUM_IN), jnp.float32),
          pltpu.VMEM((_LCAP,), jnp.int32),
          pltpu.VMEM((_LCAP,), jnp.int32),
          pltpu.VMEM((_GK,), jnp.int32),
          pltpu.VMEM((_GK,), jnp.int32),
          pltpu.VMEM((_GK,), jnp.float32),
          pltpu.VMEM((_GK, NUM_IN), jnp.float32),
          pltpu.VMEM((_GK,), jnp.int32),
          pltpu.VMEM((_GK,), jnp.int32),
          pltpu.VMEM((_GK,), jnp.float32),
          pltpu.VMEM((_GK, NUM_IN), jnp.float32),
          pltpu.VMEM((_HOP_W,), jnp.int32),
          pltpu.VMEM((_HOP_W,), jnp.int32),
          pltpu.VMEM((_WPAD + 16,), jnp.float32),
          pltpu.SemaphoreType.DMA,
          pltpu.SemaphoreType.DMA,
          pltpu.SemaphoreType.DMA,
      ],
  )
  return f(src, index, w64pad)


_POOL_W = 2048
_ROWS_PER_TILE = N_NODES // (NC * NS)
_G_PER_TILE = N_GRAPHS // NS
LG = 16


_PACC = N_GRAPHS * LG
_ZP = _PACC // NS


def _pool_body(lg_hbm, batch_hbm, out_hbm, pacc, zbuf, rbuf, bbuf, ibuf):
  c = lax.axis_index("c")
  s = lax.axis_index("s")
  lane = lax.iota(jnp.int32, L)

  def zero_vec(i, _):
    zbuf[pl.ds(i * L, L)] = jnp.zeros((L,), jnp.float32)
    return 0
  lax.fori_loop(0, _ZP // L, zero_vec, 0)
  pltpu.sync_copy(zbuf, pacc.at[pl.ds(s * _ZP, _ZP)])
  plsc.subcore_barrier()

  tile_base = (c * NS + s) * _ROWS_PER_TILE

  def win(w, _):
    rbase = tile_base + w * _POOL_W
    pltpu.sync_copy(lg_hbm.at[pl.ds(rbase * LG, _POOL_W * LG)], rbuf)
    pltpu.sync_copy(batch_hbm.at[pl.ds(rbase, _POOL_W)], bbuf)

    mall = lane >= 0
    def expand(g, _):
      bv = bbuf[pl.ds(g * L, L)] * LG
      ppos = g * (L * LG) + lane * LG
      for j in range(LG):
        plsc.store_scatter(ibuf, [ppos + j], bv + j, mask=mall)
      return 0
    lax.fori_loop(0, _POOL_W // L, expand, 0)
    pltpu.sync_copy(rbuf, pacc.at[ibuf], add=True)
    return 0
  lax.fori_loop(0, _ROWS_PER_TILE // _POOL_W, win, 0)
  plsc.subcore_barrier()

  pltpu.sync_copy(pacc.at[pl.ds(s * _ZP, _ZP)],
                  out_hbm.at[c, pl.ds(s * _ZP, _ZP)])


def _k_pool(lg_flat, batch):
  f = pl.kernel(
      _pool_body,
      out_type=jax.ShapeDtypeStruct((NC, _PACC), jnp.float32),
      mesh=plsc.VectorSubcoreMesh(**_MESH),
      compiler_params=pltpu.CompilerParams(needs_layout_passes=False),
      scratch_types=[
          pltpu.VMEM_SHARED((_PACC,), jnp.float32),
          pltpu.VMEM((_ZP,), jnp.float32),
          pltpu.VMEM((_POOL_W * LG,), jnp.float32),
          pltpu.VMEM((_POOL_W,), jnp.int32),
          pltpu.VMEM((_POOL_W * LG,), jnp.int32),
      ],
  )
  return f(lg_flat, batch)


_PRE_B = 2048
_NR = N_NODES // NUM_IN
_PRE_R = _PRE_B // NUM_IN


def _pre_body(degp_ref, x_ref, dinv_ref, dinv2_ref, y0_ref):
  deg = 1.0 + degp_ref[0] + degp_ref[1]
  dinv = lax.rsqrt(deg)
  dinv_ref[...] = dinv
  dinv2_ref[...] = 1.0 / deg
  y0_ref[...] = x_ref[...] * dinv[:, :, None]


def _k_pre(degp, x):
  grid = N_NODES // _PRE_B
  return pl.pallas_call(
      _pre_body,
      grid=(grid,),
      in_specs=[
          pl.BlockSpec((NC, _PRE_R, NUM_IN), lambda i: (0, i, 0)),
          pl.BlockSpec((_PRE_R, NUM_IN, NUM_IN), lambda i: (i, 0, 0)),
      ],
      out_specs=[
          pl.BlockSpec((_PRE_R, NUM_IN), lambda i: (i, 0)),
          pl.BlockSpec((_PRE_R, NUM_IN), lambda i: (i, 0)),
          pl.BlockSpec((_PRE_R, NUM_IN, NUM_IN), lambda i: (i, 0, 0)),
      ],
      out_shape=[
          jax.ShapeDtypeStruct((_NR, NUM_IN), jnp.float32),
          jax.ShapeDtypeStruct((_NR, NUM_IN), jnp.float32),
          jax.ShapeDtypeStruct((_NR, NUM_IN, NUM_IN), jnp.float32),
      ],
  )(degp.reshape(NC, _NR, NUM_IN), x.reshape(_NR, NUM_IN, NUM_IN))


def _scale_body(y_ref, d_ref, o_ref):
  o_ref[...] = y_ref[...] * d_ref[...][:, :, None]


def _k_scale(y3d, d2d):
  grid = _NR // _PRE_R
  return pl.pallas_call(
      _scale_body,
      grid=(grid,),
      in_specs=[
          pl.BlockSpec((_PRE_R, NUM_IN, NUM_IN), lambda i: (i, 0, 0)),
          pl.BlockSpec((_PRE_R, NUM_IN), lambda i: (i, 0)),
      ],
      out_specs=pl.BlockSpec((_PRE_R, NUM_IN, NUM_IN), lambda i: (i, 0, 0)),
      out_shape=jax.ShapeDtypeStruct((_NR, NUM_IN, NUM_IN), jnp.float32),
  )(y3d, d2d)


_MM_B = 2048
_MM_R = _MM_B // NUM_IN


def _mm_body(y3_ref, dinv_ref, linwT_ref, linb_ref, fcwT_ref, o_ref):
  h2 = (y3_ref[...] * dinv_ref[...][:, :, None]).reshape(_MM_B, NUM_IN)
  h = jnp.dot(h2, linwT_ref[...], preferred_element_type=jnp.float32)
  h = jnp.maximum(h + linb_ref[...], 0.0)
  o_ref[...] = jnp.dot(h, fcwT_ref[...], preferred_element_type=jnp.float32)


def _k_mm(y3_3d, dinv2d, linwT, linb, fcwT):
  grid = N_NODES // _MM_B
  return pl.pallas_call(
      _mm_body,
      grid=(grid,),
      in_specs=[
          pl.BlockSpec((_MM_R, NUM_IN, NUM_IN), lambda i: (i, 0, 0)),
          pl.BlockSpec((_MM_R, NUM_IN), lambda i: (i, 0)),
          pl.BlockSpec((NUM_IN, NUM_HIDDEN), lambda i: (0, 0)),
          pl.BlockSpec((1, NUM_HIDDEN), lambda i: (0, 0)),
          pl.BlockSpec((NUM_HIDDEN, LG), lambda i: (0, 0)),
      ],
      out_specs=pl.BlockSpec((_MM_B, LG), lambda i: (i, 0)),
      out_shape=jax.ShapeDtypeStruct((N_NODES, LG), jnp.float32),
  )(y3_3d, dinv2d, linwT, linb, fcwT)


def _soft_body(pp_ref, fcb_ref, o_ref):
  z = pp_ref[0] + pp_ref[1] + fcb_ref[...]
  col = lax.broadcasted_iota(jnp.int32, (N_GRAPHS, LG), 1)
  valid = col < NUM_CLASS
  z = jnp.where(valid, z, -1e30)
  z = z - jnp.max(z, axis=1, keepdims=True)
  p = jnp.exp(z)
  p = jnp.where(valid, p, 0.0)
  o_ref[...] = p / jnp.sum(p, axis=1, keepdims=True)


def _k_soft(pooledp, fcb):
  return pl.pallas_call(
      _soft_body,
      in_specs=[
          pl.BlockSpec((NC, N_GRAPHS, LG), lambda: (0, 0, 0)),
          pl.BlockSpec((1, LG), lambda: (0, 0)),
      ],
      out_specs=pl.BlockSpec((N_GRAPHS, LG), lambda: (0, 0)),
      out_shape=jax.ShapeDtypeStruct((N_GRAPHS, LG), jnp.float32),
  )(pooledp, fcb)


def kernel(x, index, batch, weight, lin_w, lin_b, fc_w, fc_b):
  index = index.astype(jnp.int32)
  batch = batch.astype(jnp.int32)
  ew_win = jnp.tile(weight, _DEG_W // E_PER_GRAPH)
  w64pad = jnp.zeros((_WPAD + 16,), jnp.float32).at[:E_PER_GRAPH].set(weight)

  degp = _k_deg(index, ew_win)
  dinv, dinv2, y0_3d = _k_pre(degp, x)
  y1 = _k_hop(y0_3d.reshape(N_NODES, NUM_IN), index, w64pad)
  y2_3d = _k_scale(y1.reshape(_NR, NUM_IN, NUM_IN), dinv2)
  y3 = _k_hop(y2_3d.reshape(N_NODES, NUM_IN), index, w64pad)

  linwT = lin_w.T
  linb = lin_b.reshape(1, NUM_HIDDEN)
  fcwT = jnp.zeros((NUM_HIDDEN, LG), jnp.float32).at[:, :NUM_CLASS].set(fc_w.T)
  lg = _k_mm(y3.reshape(_NR, NUM_IN, NUM_IN), dinv, linwT, linb, fcwT)

  pooledp = _k_pool(lg.reshape(-1), batch)
  fcb = jnp.zeros((1, LG), jnp.float32).at[0, :NUM_CLASS].set(fc_b)
  probs = _k_soft(pooledp.reshape(NC, N_GRAPHS, LG), fcb)
  return probs[:, :NUM_CLASS]


# speedup vs baseline: 1.9732x; 1.1286x over previous
import functools

import jax
import jax.numpy as jnp
from jax import lax
from jax.experimental import pallas as pl
from jax.experimental.pallas import tpu as pltpu
import jax.experimental.pallas.tpu_sc as plsc

N_NODES = 262144
N_EDGES = 524288
NUM_IN = 128
NUM_HIDDEN = 256
NUM_CLASS = 10
N_GRAPHS = 8192
E_PER_GRAPH = 64

NC = 2
NS = 16
L = 16

_MESH = dict(core_axis_name="c", subcore_axis_name="s", num_cores=NC,
             num_subcores=NS)

_DEG_W = 4096
_E_PER_TILE_DEG = N_EDGES // (NC * NS)
_N_PER_TILE = N_NODES // NS


def _deg_body(index_hbm, ew_win_hbm, degp_hbm, dacc, zbuf, colbuf, ewb):
  c = lax.axis_index("c")
  s = lax.axis_index("s")

  def zero_vec(i, _):
    zbuf[pl.ds(i * L, L)] = jnp.zeros((L,), jnp.float32)
    return 0
  lax.fori_loop(0, _DEG_W // L, zero_vec, 0)

  def zero_chunk(i, _):
    pltpu.sync_copy(zbuf, dacc.at[pl.ds(s * _N_PER_TILE + i * _DEG_W, _DEG_W)])
    return 0
  lax.fori_loop(0, _N_PER_TILE // _DEG_W, zero_chunk, 0)
  plsc.subcore_barrier()

  pltpu.sync_copy(ew_win_hbm, ewb)
  tile_base = (c * NS + s) * _E_PER_TILE_DEG

  def win(w, _):
    ebase = tile_base + w * _DEG_W
    pltpu.sync_copy(index_hbm.at[1, pl.ds(ebase, _DEG_W)], colbuf)
    pltpu.sync_copy(ewb, dacc.at[colbuf], add=True)
    return 0
  lax.fori_loop(0, _E_PER_TILE_DEG // _DEG_W, win, 0)
  plsc.subcore_barrier()

  pltpu.sync_copy(dacc.at[pl.ds(s * _N_PER_TILE, _N_PER_TILE)],
                  degp_hbm.at[c, pl.ds(s * _N_PER_TILE, _N_PER_TILE)])


def _k_deg(index, ew_win):
  f = pl.kernel(
      _deg_body,
      out_type=jax.ShapeDtypeStruct((NC, N_NODES), jnp.float32),
      mesh=plsc.VectorSubcoreMesh(**_MESH),
      compiler_params=pltpu.CompilerParams(needs_layout_passes=False),
      scratch_types=[
          pltpu.VMEM_SHARED((N_NODES,), jnp.float32),
          pltpu.VMEM((_DEG_W,), jnp.float32),
          pltpu.VMEM((_DEG_W,), jnp.int32),
          pltpu.VMEM((_DEG_W,), jnp.float32),
      ],
  )
  return f(index, ew_win)


_R = 10240
_CHUNKS = 26
_PASSES = _CHUNKS // NC
_HOP_W = 2048
_UNROLL = 1
_E_PER_TILE = N_EDGES // NS
_GK = 128
_LCAP = 2208
_RPT = _R // NS
_TAIL_BASE = (_CHUNKS - 1) * _R
_TAIL_ROWS = N_NODES - _TAIL_BASE
_TAIL_RPT = _TAIL_ROWS // NS
_WPAD = 64
_LCBITS = 14
_LCMASK = (1 << _LCBITS) - 1


def _hop_stage(flr, flc, st_r, st_c, st_w, ewb, start, mall):
  def stage(i, _):
    st_r[pl.ds(i * L, L)] = flr[pl.ds(start + i * L, L)]
    pk = flc[pl.ds(start + i * L, L)]
    st_c[pl.ds(i * L, L)] = pk & _LCMASK
    st_w[pl.ds(i * L, L)] = plsc.load_gather(
        ewb, [lax.shift_right_logical(pk, _LCBITS)], mask=mall)
    return 0
  lax.fori_loop(0, _GK // L, stage, 0)


def _hop_finish(src_hbm, acc, st_r, st_c, st_w, gbuf, gsem):
  pltpu.make_async_copy(src_hbm.at[st_r], gbuf, gsem).wait()

  def scale(g, _):
    wv = st_w[pl.ds(g * L, L)]
    for k in range(L):
      wsp = jnp.full((L,), wv[k], jnp.float32)
      r = g * L + k
      for q in range(NUM_IN // L):
        gbuf[r, pl.ds(q * L, L)] = gbuf[r, pl.ds(q * L, L)] * wsp
    return 0
  lax.fori_loop(0, _GK // L, scale, 0)

  pltpu.sync_copy(gbuf, acc.at[st_c], add=True)


def _hop_fire(src_hbm, acc, flr, flc, stA, stB, gbufA, gbufB, ewb,
              gsemA, gsemB, start, fctr, mall):
  par = lax.rem(fctr, 2)
  @pl.when(par == 0)
  def _():
    _hop_stage(flr, flc, stA[0], stA[1], stA[2], ewb, start, mall)
    pltpu.async_copy(src_hbm.at[stA[0]], gbufA, gsemA)
  @pl.when(par == 1)
  def _():
    _hop_stage(flr, flc, stB[0], stB[1], stB[2], ewb, start, mall)
    pltpu.async_copy(src_hbm.at[stB[0]], gbufB, gsemB)
  @pl.when((fctr >= 1) & (par == 1))
  def _():
    _hop_finish(src_hbm, acc, stA[0], stA[1], stA[2], gbufA, gsemA)
  @pl.when((fctr >= 1) & (par == 0))
  def _():
    _hop_finish(src_hbm, acc, stB[0], stB[1], stB[2], gbufB, gsemB)


def _hop_body(src_hbm, index_hbm, w64_hbm, dst_hbm,
              acc, flr, flc, st_rA, st_cA, st_wA, gbufA,
              st_rB, st_cB, st_wB, gbufB, colw, roww, ewb,
              gsemA, gsemB, wsem):
  stA = (st_rA, st_cA, st_wA)
  stB = (st_rB, st_cB, st_wB)
  c = lax.axis_index("c")
  s = lax.axis_index("s")
  pltpu.sync_copy(w64_hbm, ewb)
  lane = lax.iota(jnp.int32, L)
  mall = lane >= 0
  tile_e = s * _E_PER_TILE
  n_win = _E_PER_TILE // _HOP_W

  def do_pass(p, _):
    k = NC * p + c
    is_tail = k == (_CHUNKS - 1)
    base = jnp.where(is_tail, _TAIL_BASE, k * _R)
    init_base = jnp.where(is_tail, N_NODES - _R, k * _R)
    hi = jnp.where(is_tail, N_NODES, k * _R + _R)

    pltpu.sync_copy(src_hbm.at[pl.ds(init_base + s * _RPT, _RPT)],
                    acc.at[pl.ds(s * _RPT, _RPT)])
    plsc.subcore_barrier()

    def start_win(w):
      ebase = tile_e + w * _HOP_W
      off = lax.rem(w, 2) * _HOP_W
      pltpu.async_copy(index_hbm.at[1, pl.ds(ebase, _HOP_W)],
                       colw.at[pl.ds(off, _HOP_W)], wsem)
      pltpu.async_copy(index_hbm.at[0, pl.ds(ebase, _HOP_W)],
                       roww.at[pl.ds(off, _HOP_W)], wsem)
    start_win(0)

    def drain(cnt, fired, fctr):
      def one(d, carry):
        fired, fctr = carry
        go = fired + _GK <= cnt
        @pl.when(go)
        def _():
          _hop_fire(src_hbm, acc, flr, flc, stA, stB, gbufA, gbufB,
                    ewb, gsemA, gsemB, fired, fctr, mall)
        adv = jnp.where(go, 1, 0)
        return (fired + adv * _GK, fctr + adv)
      fired, fctr = lax.fori_loop(0, (_LCAP + _GK - 1) // _GK, one,
                                  (fired, fctr))
      rem = cnt - fired
      def mv(i, _):
        @pl.when(i * L < rem)
        def _():
          flr[pl.ds(i * L, L)] = flr[pl.ds(fired + i * L, L)]
          flc[pl.ds(i * L, L)] = flc[pl.ds(fired + i * L, L)]
        return 0
      lax.fori_loop(0, _GK // L, mv, 0)
      return rem, fctr

    def win(w, carry):
      cnt, fctr = carry
      ebase = tile_e + w * _HOP_W
      @pl.when(w + 1 < n_win)
      def _():
        start_win(w + 1)
      off = lax.rem(w, 2) * _HOP_W
      pltpu.make_async_copy(index_hbm.at[1, pl.ds(ebase, _HOP_W)],
                            colw.at[pl.ds(off, _HOP_W)], wsem).wait()
      pltpu.make_async_copy(index_hbm.at[0, pl.ds(ebase, _HOP_W)],
                            roww.at[pl.ds(off, _HOP_W)], wsem).wait()

      def scan(jj, cnt):
        for u in range(_UNROLL):
          j = jj * _UNROLL + u
          c16 = colw[pl.ds(off + j * L, L)]
          m = (c16 >= base) & (c16 < hi)
          r16 = roww[pl.ds(off + j * L, L)]
          widx16 = lax.rem(j, 4) * L + lane
          pk16 = (c16 - init_base) | lax.shift_left(widx16, _LCBITS)
          cs = plsc.cumsum(m.astype(jnp.int32))
          pos = cs + (cnt - 1)
          plsc.store_scatter(flr, [pos], r16, mask=m)
          plsc.store_scatter(flc, [pos], pk16, mask=m)
          cnt = cnt + cs[L - 1]
        return cnt
      cnt = lax.fori_loop(0, _HOP_W // L // _UNROLL, scan, cnt)
      return drain(cnt, 0, fctr)

    cnt, fctr = lax.fori_loop(0, n_win, win, (0, 0))

    cnt_pad = jnp.where(cnt > 0, ((cnt + _GK - 1) // _GK) * _GK, 0)
    pad_pk = lane | (_WPAD << _LCBITS)
    def pad(i, _):
      @pl.when(cnt + i * L < cnt_pad)
      def _():
        off = cnt + i * L
        flr[pl.ds(off, L)] = lane
        flc[pl.ds(off, L)] = pad_pk
      return 0
    lax.fori_loop(0, _GK // L, pad, 0)
    _, fctr = drain(cnt_pad, 0, fctr)

    last = lax.rem(fctr - 1, 2)
    @pl.when((fctr >= 1) & (last == 0))
    def _():
      _hop_finish(src_hbm, acc, stA[0], stA[1], stA[2], gbufA, gsemA)
    @pl.when((fctr >= 1) & (last == 1))
    def _():
      _hop_finish(src_hbm, acc, stB[0], stB[1], stB[2], gbufB, gsemB)

    plsc.subcore_barrier()
    @pl.when(jnp.logical_not(is_tail))
    def _():
      pltpu.sync_copy(acc.at[pl.ds(s * _RPT, _RPT)],
                      dst_hbm.at[pl.ds(init_base + s * _RPT, _RPT)])
    @pl.when(is_tail)
    def _():
      pltpu.sync_copy(
          acc.at[pl.ds(_R - _TAIL_ROWS + s * _TAIL_RPT, _TAIL_RPT)],
          dst_hbm.at[pl.ds(_TAIL_BASE + s * _TAIL_RPT, _TAIL_RPT)])
    plsc.subcore_barrier()
    return 0

  lax.fori_loop(0, _PASSES, do_pass, 0)


def _k_hop(src, index, w64pad):
  f = pl.kernel(
      _hop_body,
      out_type=jax.ShapeDtypeStruct((N_NODES, NUM_IN), jnp.float32),
      mesh=plsc.VectorSubcoreMesh(**_MESH),
      compiler_params=pltpu.CompilerParams(needs_layout_passes=False),
      scratch_types=[
          pltpu.VMEM_SHARED((_R, NUM_IN), jnp.float32),
          pltpu.VMEM((_LCAP,), jnp.int32),
          pltpu.VMEM((_LCAP,), jnp.int32),
          pltpu.VMEM((_GK,), jnp.int32),
          pltpu.VMEM((_GK,), jnp.int32),
          pltpu.VMEM((_GK,), jnp.float32),
          pltpu.VMEM((_GK, NUM_IN), jnp.float32),
          pltpu.VMEM((_GK,), jnp.int32),
          pltpu.VMEM((_GK,), jnp.int32),
          pltpu.VMEM((_GK,), jnp.float32),
          pltpu.VMEM((_GK, NUM_IN), jnp.float32),
          pltpu.VMEM((2 * _HOP_W,), jnp.int32),
          pltpu.VMEM((2 * _HOP_W,), jnp.int32),
          pltpu.VMEM((_WPAD + 16,), jnp.float32),
          pltpu.SemaphoreType.DMA,
          pltpu.SemaphoreType.DMA,
          pltpu.SemaphoreType.DMA,
      ],
  )
  return f(src, index, w64pad)


_POOL_W = 2048
_ROWS_PER_TILE = N_NODES // (NC * NS)
_G_PER_TILE = N_GRAPHS // NS
LG = 16


_PACC = N_GRAPHS * LG
_ZP = _PACC // NS


def _pool_body(lg_hbm, batch_hbm, out_hbm, pacc, zbuf, rbuf, bbuf, ibuf):
  c = lax.axis_index("c")
  s = lax.axis_index("s")
  lane = lax.iota(jnp.int32, L)

  def zero_vec(i, _):
    zbuf[pl.ds(i * L, L)] = jnp.zeros((L,), jnp.float32)
    return 0
  lax.fori_loop(0, _ZP // L, zero_vec, 0)
  pltpu.sync_copy(zbuf, pacc.at[pl.ds(s * _ZP, _ZP)])
  plsc.subcore_barrier()

  tile_base = (c * NS + s) * _ROWS_PER_TILE

  def win(w, _):
    rbase = tile_base + w * _POOL_W
    pltpu.sync_copy(lg_hbm.at[pl.ds(rbase * LG, _POOL_W * LG)], rbuf)
    pltpu.sync_copy(batch_hbm.at[pl.ds(rbase, _POOL_W)], bbuf)

    mall = lane >= 0
    def expand(g, _):
      bv = bbuf[pl.ds(g * L, L)] * LG
      ppos = g * (L * LG) + lane * LG
      for j in range(LG):
        plsc.store_scatter(ibuf, [ppos + j], bv + j, mask=mall)
      return 0
    lax.fori_loop(0, _POOL_W // L, expand, 0)
    pltpu.sync_copy(rbuf, pacc.at[ibuf], add=True)
    return 0
  lax.fori_loop(0, _ROWS_PER_TILE // _POOL_W, win, 0)
  plsc.subcore_barrier()

  pltpu.sync_copy(pacc.at[pl.ds(s * _ZP, _ZP)],
                  out_hbm.at[c, pl.ds(s * _ZP, _ZP)])


def _k_pool(lg_flat, batch):
  f = pl.kernel(
      _pool_body,
      out_type=jax.ShapeDtypeStruct((NC, _PACC), jnp.float32),
      mesh=plsc.VectorSubcoreMesh(**_MESH),
      compiler_params=pltpu.CompilerParams(needs_layout_passes=False),
      scratch_types=[
          pltpu.VMEM_SHARED((_PACC,), jnp.float32),
          pltpu.VMEM((_ZP,), jnp.float32),
          pltpu.VMEM((_POOL_W * LG,), jnp.float32),
          pltpu.VMEM((_POOL_W,), jnp.int32),
          pltpu.VMEM((_POOL_W * LG,), jnp.int32),
      ],
  )
  return f(lg_flat, batch)


_PRE_B = 2048
_NR = N_NODES // NUM_IN
_PRE_R = _PRE_B // NUM_IN


def _pre_body(degp_ref, x_ref, dinv_ref, dinv2_ref, y0_ref):
  deg = 1.0 + degp_ref[0] + degp_ref[1]
  dinv = lax.rsqrt(deg)
  dinv_ref[...] = dinv
  dinv2_ref[...] = 1.0 / deg
  y0_ref[...] = x_ref[...] * dinv[:, :, None]


def _k_pre(degp, x):
  grid = N_NODES // _PRE_B
  return pl.pallas_call(
      _pre_body,
      grid=(grid,),
      in_specs=[
          pl.BlockSpec((NC, _PRE_R, NUM_IN), lambda i: (0, i, 0)),
          pl.BlockSpec((_PRE_R, NUM_IN, NUM_IN), lambda i: (i, 0, 0)),
      ],
      out_specs=[
          pl.BlockSpec((_PRE_R, NUM_IN), lambda i: (i, 0)),
          pl.BlockSpec((_PRE_R, NUM_IN), lambda i: (i, 0)),
          pl.BlockSpec((_PRE_R, NUM_IN, NUM_IN), lambda i: (i, 0, 0)),
      ],
      out_shape=[
          jax.ShapeDtypeStruct((_NR, NUM_IN), jnp.float32),
          jax.ShapeDtypeStruct((_NR, NUM_IN), jnp.float32),
          jax.ShapeDtypeStruct((_NR, NUM_IN, NUM_IN), jnp.float32),
      ],
  )(degp.reshape(NC, _NR, NUM_IN), x.reshape(_NR, NUM_IN, NUM_IN))


def _scale_body(y_ref, d_ref, o_ref):
  o_ref[...] = y_ref[...] * d_ref[...][:, :, None]


def _k_scale(y3d, d2d):
  grid = _NR // _PRE_R
  return pl.pallas_call(
      _scale_body,
      grid=(grid,),
      in_specs=[
          pl.BlockSpec((_PRE_R, NUM_IN, NUM_IN), lambda i: (i, 0, 0)),
          pl.BlockSpec((_PRE_R, NUM_IN), lambda i: (i, 0)),
      ],
      out_specs=pl.BlockSpec((_PRE_R, NUM_IN, NUM_IN), lambda i: (i, 0, 0)),
      out_shape=jax.ShapeDtypeStruct((_NR, NUM_IN, NUM_IN), jnp.float32),
  )(y3d, d2d)


_MM_B = 2048
_MM_R = _MM_B // NUM_IN


def _mm_body(y3_ref, dinv_ref, linwT_ref, linb_ref, fcwT_ref, o_ref):
  h2 = (y3_ref[...] * dinv_ref[...][:, :, None]).reshape(_MM_B, NUM_IN)
  h = jnp.dot(h2, linwT_ref[...], preferred_element_type=jnp.float32)
  h = jnp.maximum(h + linb_ref[...], 0.0)
  o_ref[...] = jnp.dot(h, fcwT_ref[...], preferred_element_type=jnp.float32)


def _k_mm(y3_3d, dinv2d, linwT, linb, fcwT):
  grid = N_NODES // _MM_B
  return pl.pallas_call(
      _mm_body,
      grid=(grid,),
      in_specs=[
          pl.BlockSpec((_MM_R, NUM_IN, NUM_IN), lambda i: (i, 0, 0)),
          pl.BlockSpec((_MM_R, NUM_IN), lambda i: (i, 0)),
          pl.BlockSpec((NUM_IN, NUM_HIDDEN), lambda i: (0, 0)),
          pl.BlockSpec((1, NUM_HIDDEN), lambda i: (0, 0)),
          pl.BlockSpec((NUM_HIDDEN, LG), lambda i: (0, 0)),
      ],
      out_specs=pl.BlockSpec((_MM_B, LG), lambda i: (i, 0)),
      out_shape=jax.ShapeDtypeStruct((N_NODES, LG), jnp.float32),
  )(y3_3d, dinv2d, linwT, linb, fcwT)


def _soft_body(pp_ref, fcb_ref, o_ref):
  z = pp_ref[0] + pp_ref[1] + fcb_ref[...]
  col = lax.broadcasted_iota(jnp.int32, (N_GRAPHS, LG), 1)
  valid = col < NUM_CLASS
  z = jnp.where(valid, z, -1e30)
  z = z - jnp.max(z, axis=1, keepdims=True)
  p = jnp.exp(z)
  p = jnp.where(valid, p, 0.0)
  o_ref[...] = p / jnp.sum(p, axis=1, keepdims=True)


def _k_soft(pooledp, fcb):
  return pl.pallas_call(
      _soft_body,
      in_specs=[
          pl.BlockSpec((NC, N_GRAPHS, LG), lambda: (0, 0, 0)),
          pl.BlockSpec((1, LG), lambda: (0, 0)),
      ],
      out_specs=pl.BlockSpec((N_GRAPHS, LG), lambda: (0, 0)),
      out_shape=jax.ShapeDtypeStruct((N_GRAPHS, LG), jnp.float32),
  )(pooledp, fcb)


def kernel(x, index, batch, weight, lin_w, lin_b, fc_w, fc_b):
  index = index.astype(jnp.int32)
  batch = batch.astype(jnp.int32)
  ew_win = jnp.tile(weight, _DEG_W // E_PER_GRAPH)
  w64pad = jnp.zeros((_WPAD + 16,), jnp.float32).at[:E_PER_GRAPH].set(weight)

  degp = _k_deg(index, ew_win)
  dinv, dinv2, y0_3d = _k_pre(degp, x)
  y1 = _k_hop(y0_3d.reshape(N_NODES, NUM_IN), index, w64pad)
  y2_3d = _k_scale(y1.reshape(_NR, NUM_IN, NUM_IN), dinv2)
  y3 = _k_hop(y2_3d.reshape(N_NODES, NUM_IN), index, w64pad)

  linwT = lin_w.T
  linb = lin_b.reshape(1, NUM_HIDDEN)
  fcwT = jnp.zeros((NUM_HIDDEN, LG), jnp.float32).at[:, :NUM_CLASS].set(fc_w.T)
  lg = _k_mm(y3.reshape(_NR, NUM_IN, NUM_IN), dinv, linwT, linb, fcwT)

  pooledp = _k_pool(lg.reshape(-1), batch)
  fcb = jnp.zeros((1, LG), jnp.float32).at[0, :NUM_CLASS].set(fc_b)
  probs = _k_soft(pooledp.reshape(NC, N_GRAPHS, LG), fcb)
  return probs[:, :NUM_CLASS]


# speedup vs baseline: 2.0264x; 1.0269x over previous
import functools

import jax
import jax.numpy as jnp
from jax import lax
from jax.experimental import pallas as pl
from jax.experimental.pallas import tpu as pltpu
import jax.experimental.pallas.tpu_sc as plsc

N_NODES = 262144
N_EDGES = 524288
NUM_IN = 128
NUM_HIDDEN = 256
NUM_CLASS = 10
N_GRAPHS = 8192
E_PER_GRAPH = 64

NC = 2
NS = 16
L = 16

_MESH = dict(core_axis_name="c", subcore_axis_name="s", num_cores=NC,
             num_subcores=NS)

_DEG_W = 4096
_E_PER_TILE_DEG = N_EDGES // (NC * NS)
_N_PER_TILE = N_NODES // NS


def _deg_body(index_hbm, ew_win_hbm, degp_hbm, dacc, zbuf, colbuf, ewb):
  c = lax.axis_index("c")
  s = lax.axis_index("s")

  def zero_vec(i, _):
    zbuf[pl.ds(i * L, L)] = jnp.zeros((L,), jnp.float32)
    return 0
  lax.fori_loop(0, _DEG_W // L, zero_vec, 0)

  def zero_chunk(i, _):
    pltpu.sync_copy(zbuf, dacc.at[pl.ds(s * _N_PER_TILE + i * _DEG_W, _DEG_W)])
    return 0
  lax.fori_loop(0, _N_PER_TILE // _DEG_W, zero_chunk, 0)
  plsc.subcore_barrier()

  pltpu.sync_copy(ew_win_hbm, ewb)
  tile_base = (c * NS + s) * _E_PER_TILE_DEG

  def win(w, _):
    ebase = tile_base + w * _DEG_W
    pltpu.sync_copy(index_hbm.at[1, pl.ds(ebase, _DEG_W)], colbuf)
    pltpu.sync_copy(ewb, dacc.at[colbuf], add=True)
    return 0
  lax.fori_loop(0, _E_PER_TILE_DEG // _DEG_W, win, 0)
  plsc.subcore_barrier()

  pltpu.sync_copy(dacc.at[pl.ds(s * _N_PER_TILE, _N_PER_TILE)],
                  degp_hbm.at[c, pl.ds(s * _N_PER_TILE, _N_PER_TILE)])


def _k_deg(index, ew_win):
  f = pl.kernel(
      _deg_body,
      out_type=jax.ShapeDtypeStruct((NC, N_NODES), jnp.float32),
      mesh=plsc.VectorSubcoreMesh(**_MESH),
      compiler_params=pltpu.CompilerParams(needs_layout_passes=False),
      scratch_types=[
          pltpu.VMEM_SHARED((N_NODES,), jnp.float32),
          pltpu.VMEM((_DEG_W,), jnp.float32),
          pltpu.VMEM((_DEG_W,), jnp.int32),
          pltpu.VMEM((_DEG_W,), jnp.float32),
      ],
  )
  return f(index, ew_win)


_R = 12288
_CHUNKS = 22
_PASSES = _CHUNKS // NC
_HOP_W = 2048
_UNROLL = 1
_E_PER_TILE = N_EDGES // NS
_GK = 64
_LCAP = 2208
_RPT = _R // NS
_TAIL_BASE = (_CHUNKS - 1) * _R
_TAIL_ROWS = N_NODES - _TAIL_BASE
_TAIL_RPT = _TAIL_ROWS // NS
_WPAD = 64
_LCBITS = 14
_LCMASK = (1 << _LCBITS) - 1


def _hop_stage(flr, flc, st_r, st_c, st_w, ewb, start, mall):
  def stage(i, _):
    st_r[pl.ds(i * L, L)] = flr[pl.ds(start + i * L, L)]
    pk = flc[pl.ds(start + i * L, L)]
    st_c[pl.ds(i * L, L)] = pk & _LCMASK
    st_w[pl.ds(i * L, L)] = plsc.load_gather(
        ewb, [lax.shift_right_logical(pk, _LCBITS)], mask=mall)
    return 0
  lax.fori_loop(0, _GK // L, stage, 0)


def _hop_finish(src_hbm, acc, st_r, st_c, st_w, gbuf, gsem):
  pltpu.make_async_copy(src_hbm.at[st_r], gbuf, gsem).wait()

  def scale(g, _):
    wv = st_w[pl.ds(g * L, L)]
    for k in range(L):
      wsp = jnp.full((L,), wv[k], jnp.float32)
      r = g * L + k
      for q in range(NUM_IN // L):
        gbuf[r, pl.ds(q * L, L)] = gbuf[r, pl.ds(q * L, L)] * wsp
    return 0
  lax.fori_loop(0, _GK // L, scale, 0)

  pltpu.sync_copy(gbuf, acc.at[st_c], add=True)


def _hop_fire(src_hbm, acc, flr, flc, stA, stB, gbufA, gbufB, ewb,
              gsemA, gsemB, start, fctr, mall):
  par = lax.rem(fctr, 2)
  @pl.when(par == 0)
  def _():
    _hop_stage(flr, flc, stA[0], stA[1], stA[2], ewb, start, mall)
    pltpu.async_copy(src_hbm.at[stA[0]], gbufA, gsemA)
  @pl.when(par == 1)
  def _():
    _hop_stage(flr, flc, stB[0], stB[1], stB[2], ewb, start, mall)
    pltpu.async_copy(src_hbm.at[stB[0]], gbufB, gsemB)
  @pl.when((fctr >= 1) & (par == 1))
  def _():
    _hop_finish(src_hbm, acc, stA[0], stA[1], stA[2], gbufA, gsemA)
  @pl.when((fctr >= 1) & (par == 0))
  def _():
    _hop_finish(src_hbm, acc, stB[0], stB[1], stB[2], gbufB, gsemB)


def _hop_body(src_hbm, index_hbm, w64_hbm, dst_hbm,
              acc, flr, flc, st_rA, st_cA, st_wA, gbufA,
              st_rB, st_cB, st_wB, gbufB, colw, roww, ewb,
              gsemA, gsemB, wsem):
  stA = (st_rA, st_cA, st_wA)
  stB = (st_rB, st_cB, st_wB)
  c = lax.axis_index("c")
  s = lax.axis_index("s")
  pltpu.sync_copy(w64_hbm, ewb)
  lane = lax.iota(jnp.int32, L)
  mall = lane >= 0
  tile_e = s * _E_PER_TILE
  n_win = _E_PER_TILE // _HOP_W

  def do_pass(p, _):
    k = NC * p + c
    is_tail = k == (_CHUNKS - 1)
    base = jnp.where(is_tail, _TAIL_BASE, k * _R)
    init_base = jnp.where(is_tail, N_NODES - _R, k * _R)
    hi = jnp.where(is_tail, N_NODES, k * _R + _R)

    pltpu.sync_copy(src_hbm.at[pl.ds(init_base + s * _RPT, _RPT)],
                    acc.at[pl.ds(s * _RPT, _RPT)])
    plsc.subcore_barrier()

    def start_win(w):
      ebase = tile_e + w * _HOP_W
      off = lax.rem(w, 2) * _HOP_W
      pltpu.async_copy(index_hbm.at[1, pl.ds(ebase, _HOP_W)],
                       colw.at[pl.ds(off, _HOP_W)], wsem)
      pltpu.async_copy(index_hbm.at[0, pl.ds(ebase, _HOP_W)],
                       roww.at[pl.ds(off, _HOP_W)], wsem)
    start_win(0)

    def drain(cnt, fired, fctr):
      def one(d, carry):
        fired, fctr = carry
        go = fired + _GK <= cnt
        @pl.when(go)
        def _():
          _hop_fire(src_hbm, acc, flr, flc, stA, stB, gbufA, gbufB,
                    ewb, gsemA, gsemB, fired, fctr, mall)
        adv = jnp.where(go, 1, 0)
        return (fired + adv * _GK, fctr + adv)
      fired, fctr = lax.fori_loop(0, (_LCAP + _GK - 1) // _GK, one,
                                  (fired, fctr))
      rem = cnt - fired
      def mv(i, _):
        @pl.when(i * L < rem)
        def _():
          flr[pl.ds(i * L, L)] = flr[pl.ds(fired + i * L, L)]
          flc[pl.ds(i * L, L)] = flc[pl.ds(fired + i * L, L)]
        return 0
      lax.fori_loop(0, _GK // L, mv, 0)
      return rem, fctr

    def win(w, carry):
      cnt, fctr = carry
      ebase = tile_e + w * _HOP_W
      @pl.when(w + 1 < n_win)
      def _():
        start_win(w + 1)
      off = lax.rem(w, 2) * _HOP_W
      pltpu.make_async_copy(index_hbm.at[1, pl.ds(ebase, _HOP_W)],
                            colw.at[pl.ds(off, _HOP_W)], wsem).wait()
      pltpu.make_async_copy(index_hbm.at[0, pl.ds(ebase, _HOP_W)],
                            roww.at[pl.ds(off, _HOP_W)], wsem).wait()

      def scan(jj, cnt):
        for u in range(_UNROLL):
          j = jj * _UNROLL + u
          c16 = colw[pl.ds(off + j * L, L)]
          m = (c16 >= base) & (c16 < hi)
          r16 = roww[pl.ds(off + j * L, L)]
          widx16 = lax.rem(j, 4) * L + lane
          pk16 = (c16 - init_base) | lax.shift_left(widx16, _LCBITS)
          cs = plsc.cumsum(m.astype(jnp.int32))
          pos = cs + (cnt - 1)
          plsc.store_scatter(flr, [pos], r16, mask=m)
          plsc.store_scatter(flc, [pos], pk16, mask=m)
          cnt = cnt + cs[L - 1]
        return cnt
      cnt = lax.fori_loop(0, _HOP_W // L // _UNROLL, scan, cnt)
      return drain(cnt, 0, fctr)

    cnt, fctr = lax.fori_loop(0, n_win, win, (0, 0))

    cnt_pad = jnp.where(cnt > 0, ((cnt + _GK - 1) // _GK) * _GK, 0)
    pad_pk = lane | (_WPAD << _LCBITS)
    def pad(i, _):
      @pl.when(cnt + i * L < cnt_pad)
      def _():
        off = cnt + i * L
        flr[pl.ds(off, L)] = lane
        flc[pl.ds(off, L)] = pad_pk
      return 0
    lax.fori_loop(0, _GK // L, pad, 0)
    _, fctr = drain(cnt_pad, 0, fctr)

    last = lax.rem(fctr - 1, 2)
    @pl.when((fctr >= 1) & (last == 0))
    def _():
      _hop_finish(src_hbm, acc, stA[0], stA[1], stA[2], gbufA, gsemA)
    @pl.when((fctr >= 1) & (last == 1))
    def _():
      _hop_finish(src_hbm, acc, stB[0], stB[1], stB[2], gbufB, gsemB)

    plsc.subcore_barrier()
    @pl.when(jnp.logical_not(is_tail))
    def _():
      pltpu.sync_copy(acc.at[pl.ds(s * _RPT, _RPT)],
                      dst_hbm.at[pl.ds(init_base + s * _RPT, _RPT)])
    @pl.when(is_tail)
    def _():
      pltpu.sync_copy(
          acc.at[pl.ds(_R - _TAIL_ROWS + s * _TAIL_RPT, _TAIL_RPT)],
          dst_hbm.at[pl.ds(_TAIL_BASE + s * _TAIL_RPT, _TAIL_RPT)])
    plsc.subcore_barrier()
    return 0

  lax.fori_loop(0, _PASSES, do_pass, 0)


def _k_hop(src, index, w64pad):
  f = pl.kernel(
      _hop_body,
      out_type=jax.ShapeDtypeStruct((N_NODES, NUM_IN), jnp.float32),
      mesh=plsc.VectorSubcoreMesh(**_MESH),
      compiler_params=pltpu.CompilerParams(needs_layout_passes=False),
      scratch_types=[
          pltpu.VMEM_SHARED((_R, NUM_IN), jnp.float32),
          pltpu.VMEM((_LCAP,), jnp.int32),
          pltpu.VMEM((_LCAP,), jnp.int32),
          pltpu.VMEM((_GK,), jnp.int32),
          pltpu.VMEM((_GK,), jnp.int32),
          pltpu.VMEM((_GK,), jnp.float32),
          pltpu.VMEM((_GK, NUM_IN), jnp.float32),
          pltpu.VMEM((_GK,), jnp.int32),
          pltpu.VMEM((_GK,), jnp.int32),
          pltpu.VMEM((_GK,), jnp.float32),
          pltpu.VMEM((_GK, NUM_IN), jnp.float32),
          pltpu.VMEM((2 * _HOP_W,), jnp.int32),
          pltpu.VMEM((2 * _HOP_W,), jnp.int32),
          pltpu.VMEM((_WPAD + 16,), jnp.float32),
          pltpu.SemaphoreType.DMA,
          pltpu.SemaphoreType.DMA,
          pltpu.SemaphoreType.DMA,
      ],
  )
  return f(src, index, w64pad)


_POOL_W = 2048
_ROWS_PER_TILE = N_NODES // (NC * NS)
_G_PER_TILE = N_GRAPHS // NS
LG = 16


_PACC = N_GRAPHS * LG
_ZP = _PACC // NS


def _pool_body(lg_hbm, batch_hbm, out_hbm, pacc, zbuf, rbuf, bbuf, ibuf):
  c = lax.axis_index("c")
  s = lax.axis_index("s")
  lane = lax.iota(jnp.int32, L)

  def zero_vec(i, _):
    zbuf[pl.ds(i * L, L)] = jnp.zeros((L,), jnp.float32)
    return 0
  lax.fori_loop(0, _ZP // L, zero_vec, 0)
  pltpu.sync_copy(zbuf, pacc.at[pl.ds(s * _ZP, _ZP)])
  plsc.subcore_barrier()

  tile_base = (c * NS + s) * _ROWS_PER_TILE

  def win(w, _):
    rbase = tile_base + w * _POOL_W
    pltpu.sync_copy(lg_hbm.at[pl.ds(rbase * LG, _POOL_W * LG)], rbuf)
    pltpu.sync_copy(batch_hbm.at[pl.ds(rbase, _POOL_W)], bbuf)

    mall = lane >= 0
    def expand(g, _):
      bv = bbuf[pl.ds(g * L, L)] * LG
      ppos = g * (L * LG) + lane * LG
      for j in range(LG):
        plsc.store_scatter(ibuf, [ppos + j], bv + j, mask=mall)
      return 0
    lax.fori_loop(0, _POOL_W // L, expand, 0)
    pltpu.sync_copy(rbuf, pacc.at[ibuf], add=True)
    return 0
  lax.fori_loop(0, _ROWS_PER_TILE // _POOL_W, win, 0)
  plsc.subcore_barrier()

  pltpu.sync_copy(pacc.at[pl.ds(s * _ZP, _ZP)],
                  out_hbm.at[c, pl.ds(s * _ZP, _ZP)])


def _k_pool(lg_flat, batch):
  f = pl.kernel(
      _pool_body,
      out_type=jax.ShapeDtypeStruct((NC, _PACC), jnp.float32),
      mesh=plsc.VectorSubcoreMesh(**_MESH),
      compiler_params=pltpu.CompilerParams(needs_layout_passes=False),
      scratch_types=[
          pltpu.VMEM_SHARED((_PACC,), jnp.float32),
          pltpu.VMEM((_ZP,), jnp.float32),
          pltpu.VMEM((_POOL_W * LG,), jnp.float32),
          pltpu.VMEM((_POOL_W,), jnp.int32),
          pltpu.VMEM((_POOL_W * LG,), jnp.int32),
      ],
  )
  return f(lg_flat, batch)


_PRE_B = 2048
_NR = N_NODES // NUM_IN
_PRE_R = _PRE_B // NUM_IN


def _pre_body(degp_ref, x_ref, dinv_ref, dinv2_ref, y0_ref):
  deg = 1.0 + degp_ref[0] + degp_ref[1]
  dinv = lax.rsqrt(deg)
  dinv_ref[...] = dinv
  dinv2_ref[...] = 1.0 / deg
  y0_ref[...] = x_ref[...] * dinv[:, :, None]


def _k_pre(degp, x):
  grid = N_NODES // _PRE_B
  return pl.pallas_call(
      _pre_body,
      grid=(grid,),
      in_specs=[
          pl.BlockSpec((NC, _PRE_R, NUM_IN), lambda i: (0, i, 0)),
          pl.BlockSpec((_PRE_R, NUM_IN, NUM_IN), lambda i: (i, 0, 0)),
      ],
      out_specs=[
          pl.BlockSpec((_PRE_R, NUM_IN), lambda i: (i, 0)),
          pl.BlockSpec((_PRE_R, NUM_IN), lambda i: (i, 0)),
          pl.BlockSpec((_PRE_R, NUM_IN, NUM_IN), lambda i: (i, 0, 0)),
      ],
      out_shape=[
          jax.ShapeDtypeStruct((_NR, NUM_IN), jnp.float32),
          jax.ShapeDtypeStruct((_NR, NUM_IN), jnp.float32),
          jax.ShapeDtypeStruct((_NR, NUM_IN, NUM_IN), jnp.float32),
      ],
  )(degp.reshape(NC, _NR, NUM_IN), x.reshape(_NR, NUM_IN, NUM_IN))


def _scale_body(y_ref, d_ref, o_ref):
  o_ref[...] = y_ref[...] * d_ref[...][:, :, None]


def _k_scale(y3d, d2d):
  grid = _NR // _PRE_R
  return pl.pallas_call(
      _scale_body,
      grid=(grid,),
      in_specs=[
          pl.BlockSpec((_PRE_R, NUM_IN, NUM_IN), lambda i: (i, 0, 0)),
          pl.BlockSpec((_PRE_R, NUM_IN), lambda i: (i, 0)),
      ],
      out_specs=pl.BlockSpec((_PRE_R, NUM_IN, NUM_IN), lambda i: (i, 0, 0)),
      out_shape=jax.ShapeDtypeStruct((_NR, NUM_IN, NUM_IN), jnp.float32),
  )(y3d, d2d)


_MM_B = 2048
_MM_R = _MM_B // NUM_IN


def _mm_body(y3_ref, dinv_ref, linwT_ref, linb_ref, fcwT_ref, o_ref):
  h2 = (y3_ref[...] * dinv_ref[...][:, :, None]).reshape(_MM_B, NUM_IN)
  h = jnp.dot(h2, linwT_ref[...], preferred_element_type=jnp.float32)
  h = jnp.maximum(h + linb_ref[...], 0.0)
  o_ref[...] = jnp.dot(h, fcwT_ref[...], preferred_element_type=jnp.float32)


def _k_mm(y3_3d, dinv2d, linwT, linb, fcwT):
  grid = N_NODES // _MM_B
  return pl.pallas_call(
      _mm_body,
      grid=(grid,),
      in_specs=[
          pl.BlockSpec((_MM_R, NUM_IN, NUM_IN), lambda i: (i, 0, 0)),
          pl.BlockSpec((_MM_R, NUM_IN), lambda i: (i, 0)),
          pl.BlockSpec((NUM_IN, NUM_HIDDEN), lambda i: (0, 0)),
          pl.BlockSpec((1, NUM_HIDDEN), lambda i: (0, 0)),
          pl.BlockSpec((NUM_HIDDEN, LG), lambda i: (0, 0)),
      ],
      out_specs=pl.BlockSpec((_MM_B, LG), lambda i: (i, 0)),
      out_shape=jax.ShapeDtypeStruct((N_NODES, LG), jnp.float32),
  )(y3_3d, dinv2d, linwT, linb, fcwT)


def _soft_body(pp_ref, fcb_ref, o_ref):
  z = pp_ref[0] + pp_ref[1] + fcb_ref[...]
  col = lax.broadcasted_iota(jnp.int32, (N_GRAPHS, LG), 1)
  valid = col < NUM_CLASS
  z = jnp.where(valid, z, -1e30)
  z = z - jnp.max(z, axis=1, keepdims=True)
  p = jnp.exp(z)
  p = jnp.where(valid, p, 0.0)
  o_ref[...] = p / jnp.sum(p, axis=1, keepdims=True)


def _k_soft(pooledp, fcb):
  return pl.pallas_call(
      _soft_body,
      in_specs=[
          pl.BlockSpec((NC, N_GRAPHS, LG), lambda: (0, 0, 0)),
          pl.BlockSpec((1, LG), lambda: (0, 0)),
      ],
      out_specs=pl.BlockSpec((N_GRAPHS, LG), lambda: (0, 0)),
      out_shape=jax.ShapeDtypeStruct((N_GRAPHS, LG), jnp.float32),
  )(pooledp, fcb)


def kernel(x, index, batch, weight, lin_w, lin_b, fc_w, fc_b):
  index = index.astype(jnp.int32)
  batch = batch.astype(jnp.int32)
  ew_win = jnp.tile(weight, _DEG_W // E_PER_GRAPH)
  w64pad = jnp.zeros((_WPAD + 16,), jnp.float32).at[:E_PER_GRAPH].set(weight)

  degp = _k_deg(index, ew_win)
  dinv, dinv2, y0_3d = _k_pre(degp, x)
  y1 = _k_hop(y0_3d.reshape(N_NODES, NUM_IN), index, w64pad)
  y2_3d = _k_scale(y1.reshape(_NR, NUM_IN, NUM_IN), dinv2)
  y3 = _k_hop(y2_3d.reshape(N_NODES, NUM_IN), index, w64pad)

  linwT = lin_w.T
  linb = lin_b.reshape(1, NUM_HIDDEN)
  fcwT = jnp.zeros((NUM_HIDDEN, LG), jnp.float32).at[:, :NUM_CLASS].set(fc_w.T)
  lg = _k_mm(y3.reshape(_NR, NUM_IN, NUM_IN), dinv, linwT, linb, fcwT)

  pooledp = _k_pool(lg.reshape(-1), batch)
  fcb = jnp.zeros((1, LG), jnp.float32).at[0, :NUM_CLASS].set(fc_b)
  probs = _k_soft(pooledp.reshape(NC, N_GRAPHS, LG), fcb)
  return probs[:, :NUM_CLASS]
